# baseline (speedup 1.0000x reference)
import functools

import jax
import jax.numpy as jnp
from jax import lax
from jax.experimental import pallas as pl
from jax.experimental.pallas import tpu as pltpu
from jax.experimental.pallas import tpu_sc as plsc

N = 10000
NP = 10240
E = 320000
EP = 327680
D_IN = 128
HEADS = 8
HID = 8
F1 = HEADS * HID
NUM_CLASSES = 40
OUT_HEADS = 8
F2 = OUT_HEADS * NUM_CLASSES
C_PAD = 48
W16 = 16

NC = 2
NS = 16
NT = NC * NS
EPT = EP // NT
CH = 128
NCHUNK = EPT // CH
CH2 = 64
NCHUNK2 = EPT // CH2
RS = NP // NS

N0C, N1C = 100, 60
NMXC = max(N0C, N1C)
TCHN = EP // CH
N0L1, N1L1 = 84, 76
N0C2, N1C2 = 184, 136
NMXC2 = max(N0C2, N1C2)
TCHN2 = EP // CH2

_MESH = dict(core_axis_name="c", subcore_axis_name="s", num_cores=NC,
             num_subcores=NS)

_f32 = jnp.float32
_i32 = jnp.int32


def _edge_pass1(src, dst, ts, td, z16):
    mesh = plsc.VectorSubcoreMesh(**_MESH)

    @functools.partial(
        pl.kernel,
        out_type=[
            jax.ShapeDtypeStruct((EP, W16), _f32),
            jax.ShapeDtypeStruct((NP, W16), _f32),
            jax.ShapeDtypeStruct((NP, W16), _f32),
        ],
        mesh=mesh,
        compiler_params=pltpu.CompilerParams(use_tc_tiling_on_sc=False),
        scratch_types=[
            pltpu.VMEM((NMXC, CH), _i32),
            pltpu.VMEM((NMXC, CH), _i32),
            pltpu.VMEM((CH, W16), _f32),
            pltpu.VMEM((CH, W16), _f32),
            pltpu.VMEM((CH, W16), _f32),
            pltpu.VMEM((CH, W16), _f32),
            pltpu.VMEM((CH, W16), _f32),
            pltpu.VMEM((CH, W16), _f32),
            pltpu.VMEM_SHARED((NP, W16), _f32),
            pltpu.SemaphoreType.DMA,
            pltpu.SemaphoreType.DMA,
            pltpu.SemaphoreType.DMA,
            pltpu.SemaphoreType.DMA,
            pltpu.SemaphoreType.DMA,
            pltpu.SemaphoreType.DMA,
            pltpu.SemaphoreType.DMA,
            pltpu.SemaphoreType.DMA,
        ],
    )
    def k(src_hbm, dst_hbm, ts_hbm, td_hbm, z_hbm, ee_hbm, da_hbm, db_hbm,
          idx_s, idx_d, sr0, sr1, dr0, dr1, eb0, eb1, dsh,
          ss0, ss1, sd0, sd1, se0, se1, sw0, sw1):
        c = lax.axis_index("c")
        s = lax.axis_index("s")
        gid0 = jnp.where(c == 0, s * N0C, NS * N0C + s * N1C)
        nch = jnp.where(c == 0, N0C, N1C)
        start = jnp.minimum(gid0, TCHN - NMXC)
        off = gid0 - start
        pltpu.sync_copy(src_hbm.at[pl.ds(start, NMXC)], idx_s)
        pltpu.sync_copy(dst_hbm.at[pl.ds(start, NMXC)], idx_d)
        pltpu.sync_copy(z_hbm.at[pl.ds(s * RS, RS)], dsh.at[pl.ds(s * RS, RS)])
        plsc.subcore_barrier()
        srows = [sr0, sr1]
        drows = [dr0, dr1]
        ebuf = [eb0, eb1]
        sems = [ss0, ss1]
        semd = [sd0, sd1]
        seme = [se0, se1]
        semw = [sw0, sw1]

        def issue(g, b):
            pltpu.async_copy(ts_hbm.at[idx_s.at[off + g]], srows[b], sems[b])
            pltpu.async_copy(td_hbm.at[idx_d.at[off + g]], drows[b], semd[b])

        issue(0, 0)
        issue(1, 1)

        @pl.loop(0, nch // 2)
        def _gg(gg):
            for b in range(2):
                g = gg * 2 + b
                pltpu.make_async_copy(ts_hbm.at[idx_s.at[off + g]], srows[b],
                                      sems[b]).wait()
                pltpu.make_async_copy(td_hbm.at[idx_d.at[off + g]], drows[b],
                                      semd[b]).wait()

                @pl.when(gg >= 1)
                def _():
                    pltpu.make_async_copy(
                        ebuf[b], ee_hbm.at[pl.ds((gid0 + g) * CH, CH)],
                        seme[b]).wait()
                    pltpu.make_async_copy(
                        ebuf[b], dsh.at[idx_d.at[off + g]], semw[b]).wait()

                for i in range(CH):
                    v = srows[b][i, :] + drows[b][i, :]
                    v = jnp.maximum(v, 0.2 * v)
                    ebuf[b][i, :] = jnp.exp(v)
                pltpu.async_copy(
                    ebuf[b], ee_hbm.at[pl.ds((gid0 + g) * CH, CH)], seme[b])
                pltpu.async_copy(
                    ebuf[b], dsh.at[idx_d.at[off + g]], semw[b], add=True)

                @pl.when(g + 2 < nch)
                def _():
                    issue(g + 2, b)

        for b in range(2):
            g_last = nch - 2 + b
            pltpu.make_async_copy(
                ebuf[b], ee_hbm.at[pl.ds((gid0 + g_last) * CH, CH)],
                seme[b]).wait()
            pltpu.make_async_copy(
                ebuf[b], dsh.at[idx_d.at[off + g_last]], semw[b]).wait()
        plsc.subcore_barrier()

        @pl.when(c == 0)
        def _():
            pltpu.sync_copy(dsh.at[pl.ds(s * RS, RS)],
                            da_hbm.at[pl.ds(s * RS, RS)])

        @pl.when(c == 1)
        def _():
            pltpu.sync_copy(dsh.at[pl.ds(s * RS, RS)],
                            db_hbm.at[pl.ds(s * RS, RS)])

    return k(src, dst, ts, td, z16)


def _edge_pass2_l1(src, dst, ee, da, db, h, z64):
    mesh = plsc.VectorSubcoreMesh(**_MESH)

    @functools.partial(
        pl.kernel,
        out_type=[
            jax.ShapeDtypeStruct((NP, F1), _f32),
            jax.ShapeDtypeStruct((NP, F1), _f32),
        ],
        mesh=mesh,
        compiler_params=pltpu.CompilerParams(use_tc_tiling_on_sc=False),
        scratch_types=[
            pltpu.VMEM((NMXC, CH), _i32),
            pltpu.VMEM((NMXC, CH), _i32),
            pltpu.VMEM((CH, W16), _f32),
            pltpu.VMEM((CH, W16), _f32),
            pltpu.VMEM((CH, W16), _f32),
            pltpu.VMEM((CH, W16), _f32),
            pltpu.VMEM((CH * W16,), _f32),
            pltpu.VMEM((CH, F1), _f32),
            pltpu.VMEM((CH, F1), _f32),
            pltpu.VMEM((CH, F1), _f32),
            pltpu.VMEM((CH, F1), _f32),
            pltpu.VMEM_SHARED((NP, F1), _f32),
            pltpu.VMEM_SHARED((NP, W16), _f32),
            pltpu.SemaphoreType.DMA,
            pltpu.SemaphoreType.DMA,
            pltpu.SemaphoreType.DMA,
            pltpu.SemaphoreType.DMA,
            pltpu.SemaphoreType.DMA,
            pltpu.SemaphoreType.DMA,
            pltpu.SemaphoreType.DMA,
            pltpu.SemaphoreType.DMA,
        ],
    )
    def k(src_hbm, dst_hbm, ee_hbm, da_hbm, db_hbm, h_hbm, z_hbm,
          oa_hbm, ob_hbm, idx_s, idx_d, eb0, eb1, dn0, dn1, albuf,
          hr0, hr1, ms0, ms1, osh, dsum,
          sh0, sh1, sd0, sd1, se0, se1, sw0, sw1):
        c = lax.axis_index("c")
        s = lax.axis_index("s")
        gid0 = jnp.where(c == 0, s * N0L1, NS * N0L1 + s * N1L1)
        nch = jnp.where(c == 0, N0L1, N1L1)
        start = jnp.minimum(gid0, TCHN - NMXC)
        off = gid0 - start
        pltpu.sync_copy(src_hbm.at[pl.ds(start, NMXC)], idx_s)
        pltpu.sync_copy(dst_hbm.at[pl.ds(start, NMXC)], idx_d)
        pltpu.sync_copy(z_hbm.at[pl.ds(s * RS, RS)], osh.at[pl.ds(s * RS, RS)])
        for r in range(RS // CH):
            row0 = s * RS + r * CH
            pltpu.sync_copy(da_hbm.at[pl.ds(row0, CH)], eb0)
            pltpu.sync_copy(db_hbm.at[pl.ds(row0, CH)], eb1)
            for i in range(CH):
                eb0[i, :] = eb0[i, :] + eb1[i, :] + 1e-16
            pltpu.sync_copy(eb0, dsum.at[pl.ds(row0, CH)])
        plsc.subcore_barrier()
        ebuf = [eb0, eb1]
        dnr = [dn0, dn1]
        hrw = [hr0, hr1]
        msg = [ms0, ms1]
        semh = [sh0, sh1]
        semd = [sd0, sd1]
        seme = [se0, se1]
        semw = [sw0, sw1]
        iomask = lax.iota(_i32, 16) < 8

        def issue(g, b):
            pltpu.async_copy(h_hbm.at[idx_s.at[off + g]], hrw[b], semh[b])
            pltpu.async_copy(dsum.at[idx_d.at[off + g]], dnr[b], semd[b])
            pltpu.async_copy(ee_hbm.at[pl.ds((gid0 + g) * CH, CH)],
                             ebuf[b], seme[b])

        issue(0, 0)
        issue(1, 1)

        @pl.loop(0, nch // 2)
        def _gg(gg):
            for b in range(2):
                g = gg * 2 + b
                pltpu.make_async_copy(h_hbm.at[idx_s.at[off + g]], hrw[b],
                                      semh[b]).wait()
                pltpu.make_async_copy(dsum.at[idx_d.at[off + g]], dnr[b],
                                      semd[b]).wait()
                pltpu.make_async_copy(ee_hbm.at[pl.ds((gid0 + g) * CH, CH)],
                                      ebuf[b], seme[b]).wait()

                @pl.when(gg >= 1)
                def _():
                    pltpu.make_async_copy(
                        msg[b], osh.at[idx_d.at[off + g]], semw[b]).wait()

                for i in range(CH):
                    albuf[pl.ds(i * W16, W16)] = ebuf[b][i, :] / dnr[b][i, :]

                @pl.loop(0, CH, unroll=4)
                def _edge(j):
                    av = albuf[pl.ds(j * W16, W16)]
                    for kq in range(4):
                        v = hrw[b][j, pl.ds(kq * 16, 16)]
                        me = jnp.full((16,), av[2 * kq], _f32)
                        mo = jnp.full((16,), av[2 * kq + 1], _f32)
                        msg[b][j, pl.ds(kq * 16, 16)] = (
                            v * jnp.where(iomask, me, mo))

                pltpu.async_copy(msg[b], osh.at[idx_d.at[off + g]], semw[b],
                                 add=True)

                @pl.when(g + 2 < nch)
                def _():
                    issue(g + 2, b)

        for b in range(2):
            g_last = nch - 2 + b
            pltpu.make_async_copy(
                msg[b], osh.at[idx_d.at[off + g_last]], semw[b]).wait()
        plsc.subcore_barrier()

        @pl.when(c == 0)
        def _():
            pltpu.sync_copy(osh.at[pl.ds(s * RS, RS)],
                            oa_hbm.at[pl.ds(s * RS, RS)])

        @pl.when(c == 1)
        def _():
            pltpu.sync_copy(osh.at[pl.ds(s * RS, RS)],
                            ob_hbm.at[pl.ds(s * RS, RS)])

    return k(src, dst, ee, da, db, h, z64)


def _edge_pass2_l2(src, dst, ee, da, db, h2, z48):
    mesh = plsc.VectorSubcoreMesh(**_MESH)

    @functools.partial(
        pl.kernel,
        out_type=[
            jax.ShapeDtypeStruct((NP, C_PAD), _f32),
            jax.ShapeDtypeStruct((NP, C_PAD), _f32),
        ],
        mesh=mesh,
        compiler_params=pltpu.CompilerParams(use_tc_tiling_on_sc=False),
        scratch_types=[
            pltpu.VMEM((NMXC2, CH2), _i32),
            pltpu.VMEM((NMXC2, CH2), _i32),
            pltpu.VMEM((CH2, W16), _f32),
            pltpu.VMEM((CH2, W16), _f32),
            pltpu.VMEM((CH2, W16), _f32),
            pltpu.VMEM((CH2, W16), _f32),
            pltpu.VMEM((CH2 * W16,), _f32),
            pltpu.VMEM((CH2, F2), _f32),
            pltpu.VMEM((CH2, F2), _f32),
            pltpu.VMEM((CH2, C_PAD), _f32),
            pltpu.VMEM((CH2, C_PAD), _f32),
            pltpu.VMEM((96,), _f32),
            pltpu.VMEM_SHARED((NP, C_PAD), _f32),
            pltpu.VMEM_SHARED((NP, W16), _f32),
            pltpu.SemaphoreType.DMA,
            pltpu.SemaphoreType.DMA,
            pltpu.SemaphoreType.DMA,
            pltpu.SemaphoreType.DMA,
            pltpu.SemaphoreType.DMA,
            pltpu.SemaphoreType.DMA,
            pltpu.SemaphoreType.DMA,
            pltpu.SemaphoreType.DMA,
        ],
    )
    def k(src_hbm, dst_hbm, ee_hbm, da_hbm, db_hbm, h_hbm, z_hbm,
          oa_hbm, ob_hbm, idx_s, idx_d, eb0, eb1, dn0, dn1, albuf,
          hr0, hr1, ms0, ms1, accbuf, osh, dsum,
          sh0, sh1, sd0, sd1, se0, se1, sw0, sw1):
        c = lax.axis_index("c")
        s = lax.axis_index("s")
        gid0 = jnp.where(c == 0, s * N0C2, NS * N0C2 + s * N1C2)
        nch = jnp.where(c == 0, N0C2, N1C2)
        start = jnp.minimum(gid0, TCHN2 - NMXC2)
        off = gid0 - start
        pltpu.sync_copy(src_hbm.at[pl.ds(start, NMXC2)], idx_s)
        pltpu.sync_copy(dst_hbm.at[pl.ds(start, NMXC2)], idx_d)
        pltpu.sync_copy(z_hbm.at[pl.ds(s * RS, RS)], osh.at[pl.ds(s * RS, RS)])
        for r in range(RS // CH2):
            row0 = s * RS + r * CH2
            pltpu.sync_copy(da_hbm.at[pl.ds(row0, CH2)], eb0)
            pltpu.sync_copy(db_hbm.at[pl.ds(row0, CH2)], eb1)
            for i in range(CH2):
                eb0[i, :] = eb0[i, :] + eb1[i, :] + 1e-16
            pltpu.sync_copy(eb0, dsum.at[pl.ds(row0, CH2)])
        accbuf[pl.ds(80, 16)] = jnp.zeros((16,), _f32)
        plsc.subcore_barrier()
        ebuf = [eb0, eb1]
        dnr = [dn0, dn1]
        hrw = [hr0, hr1]
        msg = [ms0, ms1]
        semh = [sh0, sh1]
        semd = [sd0, sd1]
        seme = [se0, se1]
        semw = [sw0, sw1]
        iomask = lax.iota(_i32, 16) < 8

        def issue(g, b):
            pltpu.async_copy(h_hbm.at[idx_s.at[off + g]], hrw[b], semh[b])
            pltpu.async_copy(dsum.at[idx_d.at[off + g]], dnr[b], semd[b])
            pltpu.async_copy(ee_hbm.at[pl.ds((gid0 + g) * CH2, CH2)],
                             ebuf[b], seme[b])

        issue(0, 0)
        issue(1, 1)

        @pl.loop(0, nch // 2)
        def _gg(gg):
            for b in range(2):
                g = gg * 2 + b
                pltpu.make_async_copy(h_hbm.at[idx_s.at[off + g]], hrw[b],
                                      semh[b]).wait()
                pltpu.make_async_copy(dsum.at[idx_d.at[off + g]], dnr[b],
                                      semd[b]).wait()
                pltpu.make_async_copy(ee_hbm.at[pl.ds((gid0 + g) * CH2, CH2)],
                                      ebuf[b], seme[b]).wait()

                @pl.when(gg >= 1)
                def _():
                    pltpu.make_async_copy(
                        msg[b], osh.at[idx_d.at[off + g]], semw[b]).wait()

                for i in range(CH2):
                    albuf[pl.ds(i * W16, W16)] = ebuf[b][i, :] / dnr[b][i, :]

                @pl.loop(0, CH2, unroll=4)
                def _edge(j):
                    av = albuf[pl.ds(j * W16, W16)]
                    acc = [jnp.zeros((16,), _f32) for _ in range(5)]
                    for p in range(4):
                        off = p * 80
                        me = jnp.full((16,), av[2 * p], _f32)
                        mo = jnp.full((16,), av[2 * p + 1], _f32)
                        mm = jnp.where(iomask, me, mo)
                        acc[0] = acc[0] + me * hrw[b][j, pl.ds(off, 16)]
                        acc[1] = acc[1] + me * hrw[b][j, pl.ds(off + 16, 16)]
                        acc[2] = acc[2] + mm * hrw[b][j, pl.ds(off + 32, 16)]
                        acc[3] = acc[3] + mo * hrw[b][j, pl.ds(off + 48, 16)]
                        acc[4] = acc[4] + mo * hrw[b][j, pl.ds(off + 64, 16)]
                    for q in range(5):
                        accbuf[pl.ds(q * 16, 16)] = acc[q]
                    msg[b][j, pl.ds(0, 16)] = (accbuf[pl.ds(0, 16)]
                                               + accbuf[pl.ds(40, 16)])
                    msg[b][j, pl.ds(16, 16)] = (accbuf[pl.ds(16, 16)]
                                                + accbuf[pl.ds(56, 16)])
                    msg[b][j, pl.ds(32, 16)] = (accbuf[pl.ds(32, 16)]
                                                + accbuf[pl.ds(72, 16)])

                pltpu.async_copy(msg[b], osh.at[idx_d.at[off + g]], semw[b],
                                 add=True)

                @pl.when(g + 2 < nch)
                def _():
                    issue(g + 2, b)

        for b in range(2):
            g_last = nch - 2 + b
            pltpu.make_async_copy(
                msg[b], osh.at[idx_d.at[off + g_last]], semw[b]).wait()
        plsc.subcore_barrier()

        @pl.when(c == 0)
        def _():
            pltpu.sync_copy(osh.at[pl.ds(s * RS, RS)],
                            oa_hbm.at[pl.ds(s * RS, RS)])

        @pl.when(c == 1)
        def _():
            pltpu.sync_copy(osh.at[pl.ds(s * RS, RS)],
                            ob_hbm.at[pl.ds(s * RS, RS)])

    return k(src, dst, ee, da, db, h2, z48)


_DOT = dict(preferred_element_type=_f32, precision=lax.Precision.HIGHEST)


def _tc1_body(x_ref, w_ref, as_ref, ad_ref, h_ref, s_ref, d_ref):
    h = jnp.dot(x_ref[...], w_ref[...], **_DOT)
    h_ref[...] = h
    a_s = jnp.dot(h, as_ref[...], **_DOT)
    a_d = jnp.dot(h, ad_ref[...], **_DOT)
    s_ref[...] = jnp.concatenate([a_s, a_d], axis=1)
    d_ref[...] = jnp.concatenate([a_d, a_s], axis=1)


def _tc1(x, w1, a_s, a_d):
    bn = 1024
    return pl.pallas_call(
        _tc1_body,
        grid=(NP // bn,),
        in_specs=[
            pl.BlockSpec((bn, D_IN), lambda i: (i, 0)),
            pl.BlockSpec((D_IN, F1), lambda i: (0, 0)),
            pl.BlockSpec((F1, HEADS), lambda i: (0, 0)),
            pl.BlockSpec((F1, HEADS), lambda i: (0, 0)),
        ],
        out_specs=[
            pl.BlockSpec((bn, F1), lambda i: (i, 0)),
            pl.BlockSpec((bn, W16), lambda i: (i, 0)),
            pl.BlockSpec((bn, W16), lambda i: (i, 0)),
        ],
        out_shape=[
            jax.ShapeDtypeStruct((NP, F1), _f32),
            jax.ShapeDtypeStruct((NP, W16), _f32),
            jax.ShapeDtypeStruct((NP, W16), _f32),
        ],
    )(x, w1, a_s, a_d)


def _tc2_body(oa_ref, ob_ref, b_ref, w_ref, as_ref, ad_ref,
              h_ref, s_ref, d_ref):
    t = oa_ref[...] + ob_ref[...] + b_ref[...]
    t = jnp.where(t > 0, t, jnp.exp(jnp.minimum(t, 0.0)) - 1.0)
    h = jnp.dot(t, w_ref[...], **_DOT)
    h_ref[...] = h
    a_s = jnp.dot(h, as_ref[...], **_DOT)
    a_d = jnp.dot(h, ad_ref[...], **_DOT)
    s_ref[...] = jnp.concatenate([a_s, a_d], axis=1)
    d_ref[...] = jnp.concatenate([a_d, a_s], axis=1)


def _tc2(oa, ob, b1, w2, a_s, a_d):
    bn = 1024
    return pl.pallas_call(
        _tc2_body,
        grid=(NP // bn,),
        in_specs=[
            pl.BlockSpec((bn, F1), lambda i: (i, 0)),
            pl.BlockSpec((bn, F1), lambda i: (i, 0)),
            pl.BlockSpec((1, F1), lambda i: (0, 0)),
            pl.BlockSpec((F1, F2), lambda i: (0, 0)),
            pl.BlockSpec((F2, HEADS), lambda i: (0, 0)),
            pl.BlockSpec((F2, HEADS), lambda i: (0, 0)),
        ],
        out_specs=[
            pl.BlockSpec((bn, F2), lambda i: (i, 0)),
            pl.BlockSpec((bn, W16), lambda i: (i, 0)),
            pl.BlockSpec((bn, W16), lambda i: (i, 0)),
        ],
        out_shape=[
            jax.ShapeDtypeStruct((NP, F2), _f32),
            jax.ShapeDtypeStruct((NP, W16), _f32),
            jax.ShapeDtypeStruct((NP, W16), _f32),
        ],
    )(oa, ob, b1, w2, a_s, a_d)


def _tc3_body(oa_ref, ob_ref, b_ref, o_ref):
    t = oa_ref[...] + ob_ref[...]
    o_ref[...] = t[:, :NUM_CLASSES] * (1.0 / OUT_HEADS) + b_ref[...]


def _tc3(oa, ob, b2):
    bn = 1000
    return pl.pallas_call(
        _tc3_body,
        grid=(N // bn,),
        in_specs=[
            pl.BlockSpec((bn, C_PAD), lambda i: (i, 0)),
            pl.BlockSpec((bn, C_PAD), lambda i: (i, 0)),
            pl.BlockSpec((1, NUM_CLASSES), lambda i: (0, 0)),
        ],
        out_specs=pl.BlockSpec((bn, NUM_CLASSES), lambda i: (i, 0)),
        out_shape=jax.ShapeDtypeStruct((N, NUM_CLASSES), _f32),
    )(oa, ob, b2)


def _expand_att(a, heads, ch):
    return jnp.repeat(jnp.eye(heads, dtype=_f32), ch, axis=0) * a.reshape(-1, 1)


def kernel(x, edge_index, W1, a_src1, a_dst1, b1, W2, a_src2, a_dst2, b2):
    src = edge_index[0].astype(_i32)
    dst = edge_index[1].astype(_i32)
    pad = jnp.full((EP - E,), N, _i32)
    src = jnp.concatenate([src, pad])
    dst = jnp.concatenate([dst, pad])
    src128 = src.reshape(EP // CH, CH)
    dst128 = dst.reshape(EP // CH, CH)
    src64 = src.reshape(EP // CH2, CH2)
    dst64 = dst.reshape(EP // CH2, CH2)

    xp = jnp.pad(x, ((0, NP - N), (0, 0)))
    as1 = _expand_att(a_src1, HEADS, HID)
    ad1 = _expand_att(a_dst1, HEADS, HID)
    as2 = _expand_att(a_src2, OUT_HEADS, NUM_CLASSES)
    ad2 = _expand_att(a_dst2, OUT_HEADS, NUM_CLASSES)

    z16 = jnp.zeros((NP, W16), _f32)
    z48 = jnp.zeros((NP, C_PAD), _f32)
    z64 = jnp.zeros((NP, F1), _f32)

    h1, ts1, td1 = _tc1(xp, W1, as1, ad1)
    ee1, da1, db1 = _edge_pass1(src128, dst128, ts1, td1, z16)
    oa1, ob1 = _edge_pass2_l1(src128, dst128, ee1, da1, db1, h1, z64)
    h2, ts2, td2 = _tc2(oa1, ob1, b1.reshape(1, F1), W2, as2, ad2)
    ee2, da2, db2 = _edge_pass1(src128, dst128, ts2, td2, z16)
    oa2, ob2 = _edge_pass2_l2(src64, dst64, ee2, da2, db2, h2, z48)
    return _tc3(oa2, ob2, b2.reshape(1, NUM_CLASSES))


# speedup vs baseline: 1.0213x; 1.0213x over previous
import functools

import jax
import jax.numpy as jnp
from jax import lax
from jax.experimental import pallas as pl
from jax.experimental.pallas import tpu as pltpu
from jax.experimental.pallas import tpu_sc as plsc

N = 10000
NP = 10240
E = 320000
EP = 327680
D_IN = 128
HEADS = 8
HID = 8
F1 = HEADS * HID
NUM_CLASSES = 40
OUT_HEADS = 8
F2 = OUT_HEADS * NUM_CLASSES
C_PAD = 48
W16 = 16

NC = 2
NS = 16
NT = NC * NS
EPT = EP // NT
CH = 128
NCHUNK = EPT // CH
CH2 = 64
NCHUNK2 = EPT // CH2
RS = NP // NS

N0C, N1C = 100, 60
NMXC = max(N0C, N1C)
TCHN = EP // CH
N0L1, N1L1 = 84, 76
N0C2, N1C2 = 184, 136
NMXC2 = max(N0C2, N1C2)
TCHN2 = EP // CH2

_MESH = dict(core_axis_name="c", subcore_axis_name="s", num_cores=NC,
             num_subcores=NS)

_f32 = jnp.float32
_i32 = jnp.int32


def _edge_pass1(src, dst, ts, td, z16):
    mesh = plsc.VectorSubcoreMesh(**_MESH)

    @functools.partial(
        pl.kernel,
        out_type=[
            jax.ShapeDtypeStruct((EP, W16), _f32),
            jax.ShapeDtypeStruct((NP, W16), _f32),
            jax.ShapeDtypeStruct((NP, W16), _f32),
        ],
        mesh=mesh,
        compiler_params=pltpu.CompilerParams(use_tc_tiling_on_sc=False),
        scratch_types=[
            pltpu.VMEM((NMXC, CH), _i32),
            pltpu.VMEM((NMXC, CH), _i32),
            pltpu.VMEM((CH, W16), _f32),
            pltpu.VMEM((CH, W16), _f32),
            pltpu.VMEM((CH, W16), _f32),
            pltpu.VMEM((CH, W16), _f32),
            pltpu.VMEM((CH, W16), _f32),
            pltpu.VMEM((CH, W16), _f32),
            pltpu.VMEM_SHARED((NP, W16), _f32),
            pltpu.SemaphoreType.DMA,
            pltpu.SemaphoreType.DMA,
            pltpu.SemaphoreType.DMA,
            pltpu.SemaphoreType.DMA,
            pltpu.SemaphoreType.DMA,
            pltpu.SemaphoreType.DMA,
            pltpu.SemaphoreType.DMA,
            pltpu.SemaphoreType.DMA,
        ],
    )
    def k(src_hbm, dst_hbm, ts_hbm, td_hbm, z_hbm, ee_hbm, da_hbm, db_hbm,
          idx_s, idx_d, sr0, sr1, dr0, dr1, eb0, eb1, dsh,
          ss0, ss1, sd0, sd1, se0, se1, sw0, sw1):
        c = lax.axis_index("c")
        s = lax.axis_index("s")
        gid0 = jnp.where(c == 0, s * N0C, NS * N0C + s * N1C)
        nch = jnp.where(c == 0, N0C, N1C)
        start = jnp.minimum(gid0, TCHN - NMXC)
        off = gid0 - start
        pltpu.sync_copy(src_hbm.at[pl.ds(start, NMXC)], idx_s)
        pltpu.sync_copy(dst_hbm.at[pl.ds(start, NMXC)], idx_d)
        pltpu.sync_copy(z_hbm.at[pl.ds(s * RS, RS)], dsh.at[pl.ds(s * RS, RS)])
        plsc.subcore_barrier()
        srows = [sr0, sr1]
        drows = [dr0, dr1]
        ebuf = [eb0, eb1]
        sems = [ss0, ss1]
        semd = [sd0, sd1]
        seme = [se0, se1]
        semw = [sw0, sw1]

        def issue(g, b):
            pltpu.async_copy(ts_hbm.at[idx_s.at[off + g]], srows[b], sems[b])
            pltpu.async_copy(td_hbm.at[idx_d.at[off + g]], drows[b], semd[b])

        issue(0, 0)
        issue(1, 1)

        @pl.loop(0, nch // 2)
        def _gg(gg):
            for b in range(2):
                g = gg * 2 + b
                pltpu.make_async_copy(ts_hbm.at[idx_s.at[off + g]], srows[b],
                                      sems[b]).wait()
                pltpu.make_async_copy(td_hbm.at[idx_d.at[off + g]], drows[b],
                                      semd[b]).wait()

                @pl.when(gg >= 1)
                def _():
                    pltpu.make_async_copy(
                        ebuf[b], ee_hbm.at[pl.ds((gid0 + g) * CH, CH)],
                        seme[b]).wait()
                    pltpu.make_async_copy(
                        ebuf[b], dsh.at[idx_d.at[off + g]], semw[b]).wait()

                for i in range(CH):
                    v = srows[b][i, :] + drows[b][i, :]
                    v = jnp.maximum(v, 0.2 * v)
                    ebuf[b][i, :] = jnp.exp(v)
                pltpu.async_copy(
                    ebuf[b], ee_hbm.at[pl.ds((gid0 + g) * CH, CH)], seme[b])
                pltpu.async_copy(
                    ebuf[b], dsh.at[idx_d.at[off + g]], semw[b], add=True)

                @pl.when(g + 2 < nch)
                def _():
                    issue(g + 2, b)

        for b in range(2):
            g_last = nch - 2 + b
            pltpu.make_async_copy(
                ebuf[b], ee_hbm.at[pl.ds((gid0 + g_last) * CH, CH)],
                seme[b]).wait()
            pltpu.make_async_copy(
                ebuf[b], dsh.at[idx_d.at[off + g_last]], semw[b]).wait()
        plsc.subcore_barrier()

        @pl.when(c == 0)
        def _():
            pltpu.sync_copy(dsh.at[pl.ds(s * RS, RS)],
                            da_hbm.at[pl.ds(s * RS, RS)])

        @pl.when(c == 1)
        def _():
            pltpu.sync_copy(dsh.at[pl.ds(s * RS, RS)],
                            db_hbm.at[pl.ds(s * RS, RS)])

    return k(src, dst, ts, td, z16)


def _edge_pass2_l1(src, dst, ee, da, db, h, z64):
    mesh = plsc.VectorSubcoreMesh(**_MESH)

    @functools.partial(
        pl.kernel,
        out_type=[
            jax.ShapeDtypeStruct((NP, F1), _f32),
            jax.ShapeDtypeStruct((NP, F1), _f32),
        ],
        mesh=mesh,
        compiler_params=pltpu.CompilerParams(use_tc_tiling_on_sc=False),
        scratch_types=[
            pltpu.VMEM((NMXC, CH), _i32),
            pltpu.VMEM((NMXC, CH), _i32),
            pltpu.VMEM((CH, W16), _f32),
            pltpu.VMEM((CH, W16), _f32),
            pltpu.VMEM((CH, W16), _f32),
            pltpu.VMEM((CH, W16), _f32),
            pltpu.VMEM((CH * W16,), _f32),
            pltpu.VMEM((CH, F1), _f32),
            pltpu.VMEM((CH, F1), _f32),
            pltpu.VMEM((CH, F1), _f32),
            pltpu.VMEM((CH, F1), _f32),
            pltpu.VMEM_SHARED((NP, F1), _f32),
            pltpu.VMEM_SHARED((NP, W16), _f32),
            pltpu.SemaphoreType.DMA,
            pltpu.SemaphoreType.DMA,
            pltpu.SemaphoreType.DMA,
            pltpu.SemaphoreType.DMA,
            pltpu.SemaphoreType.DMA,
            pltpu.SemaphoreType.DMA,
            pltpu.SemaphoreType.DMA,
            pltpu.SemaphoreType.DMA,
        ],
    )
    def k(src_hbm, dst_hbm, ee_hbm, da_hbm, db_hbm, h_hbm, z_hbm,
          oa_hbm, ob_hbm, idx_s, idx_d, eb0, eb1, dn0, dn1, albuf,
          hr0, hr1, ms0, ms1, osh, dsum,
          sh0, sh1, sd0, sd1, se0, se1, sw0, sw1):
        c = lax.axis_index("c")
        s = lax.axis_index("s")
        gid0 = jnp.where(c == 0, s * N0L1, NS * N0L1 + s * N1L1)
        nch = jnp.where(c == 0, N0L1, N1L1)
        start = jnp.minimum(gid0, TCHN - NMXC)
        off = gid0 - start
        pltpu.sync_copy(src_hbm.at[pl.ds(start, NMXC)], idx_s)
        pltpu.sync_copy(dst_hbm.at[pl.ds(start, NMXC)], idx_d)
        pltpu.sync_copy(z_hbm.at[pl.ds(s * RS, RS)], osh.at[pl.ds(s * RS, RS)])
        for r in range(RS // CH):
            row0 = s * RS + r * CH
            pltpu.sync_copy(da_hbm.at[pl.ds(row0, CH)], eb0)
            pltpu.sync_copy(db_hbm.at[pl.ds(row0, CH)], eb1)
            for i in range(CH):
                eb0[i, :] = eb0[i, :] + eb1[i, :] + 1e-16
            pltpu.sync_copy(eb0, dsum.at[pl.ds(row0, CH)])
        plsc.subcore_barrier()
        ebuf = [eb0, eb1]
        dnr = [dn0, dn1]
        hrw = [hr0, hr1]
        msg = [ms0, ms1]
        semh = [sh0, sh1]
        semd = [sd0, sd1]
        seme = [se0, se1]
        semw = [sw0, sw1]
        iomask = lax.iota(_i32, 16) < 8

        def issue(g, b):
            pltpu.async_copy(h_hbm.at[idx_s.at[off + g]], hrw[b], semh[b])
            pltpu.async_copy(dsum.at[idx_d.at[off + g]], dnr[b], semd[b])
            pltpu.async_copy(ee_hbm.at[pl.ds((gid0 + g) * CH, CH)],
                             ebuf[b], seme[b])

        issue(0, 0)
        issue(1, 1)

        @pl.loop(0, nch // 2)
        def _gg(gg):
            for b in range(2):
                g = gg * 2 + b
                pltpu.make_async_copy(h_hbm.at[idx_s.at[off + g]], hrw[b],
                                      semh[b]).wait()
                pltpu.make_async_copy(dsum.at[idx_d.at[off + g]], dnr[b],
                                      semd[b]).wait()
                pltpu.make_async_copy(ee_hbm.at[pl.ds((gid0 + g) * CH, CH)],
                                      ebuf[b], seme[b]).wait()

                @pl.when(gg >= 1)
                def _():
                    pltpu.make_async_copy(
                        msg[b], osh.at[idx_d.at[off + g]], semw[b]).wait()

                for i in range(CH):
                    albuf[pl.ds(i * W16, W16)] = ebuf[b][i, :] / dnr[b][i, :]

                @pl.loop(0, CH)
                def _edge(j):
                    av = albuf[pl.ds(j * W16, W16)]
                    for kq in range(4):
                        v = hrw[b][j, pl.ds(kq * 16, 16)]
                        me = jnp.full((16,), av[2 * kq], _f32)
                        mo = jnp.full((16,), av[2 * kq + 1], _f32)
                        msg[b][j, pl.ds(kq * 16, 16)] = (
                            v * jnp.where(iomask, me, mo))

                pltpu.async_copy(msg[b], osh.at[idx_d.at[off + g]], semw[b],
                                 add=True)

                @pl.when(g + 2 < nch)
                def _():
                    issue(g + 2, b)

        for b in range(2):
            g_last = nch - 2 + b
            pltpu.make_async_copy(
                msg[b], osh.at[idx_d.at[off + g_last]], semw[b]).wait()
        plsc.subcore_barrier()

        @pl.when(c == 0)
        def _():
            pltpu.sync_copy(osh.at[pl.ds(s * RS, RS)],
                            oa_hbm.at[pl.ds(s * RS, RS)])

        @pl.when(c == 1)
        def _():
            pltpu.sync_copy(osh.at[pl.ds(s * RS, RS)],
                            ob_hbm.at[pl.ds(s * RS, RS)])

    return k(src, dst, ee, da, db, h, z64)


def _edge_pass2_l2(src, dst, ee, da, db, h2, z48):
    mesh = plsc.VectorSubcoreMesh(**_MESH)

    @functools.partial(
        pl.kernel,
        out_type=[
            jax.ShapeDtypeStruct((NP, C_PAD), _f32),
            jax.ShapeDtypeStruct((NP, C_PAD), _f32),
        ],
        mesh=mesh,
        compiler_params=pltpu.CompilerParams(use_tc_tiling_on_sc=False),
        scratch_types=[
            pltpu.VMEM((NMXC2, CH2), _i32),
            pltpu.VMEM((NMXC2, CH2), _i32),
            pltpu.VMEM((CH2, W16), _f32),
            pltpu.VMEM((CH2, W16), _f32),
            pltpu.VMEM((CH2, W16), _f32),
            pltpu.VMEM((CH2, W16), _f32),
            pltpu.VMEM((CH2 * W16,), _f32),
            pltpu.VMEM((CH2, F2), _f32),
            pltpu.VMEM((CH2, F2), _f32),
            pltpu.VMEM((CH2, C_PAD), _f32),
            pltpu.VMEM((CH2, C_PAD), _f32),
            pltpu.VMEM((96,), _f32),
            pltpu.VMEM_SHARED((NP, C_PAD), _f32),
            pltpu.VMEM_SHARED((NP, W16), _f32),
            pltpu.SemaphoreType.DMA,
            pltpu.SemaphoreType.DMA,
            pltpu.SemaphoreType.DMA,
            pltpu.SemaphoreType.DMA,
            pltpu.SemaphoreType.DMA,
            pltpu.SemaphoreType.DMA,
            pltpu.SemaphoreType.DMA,
            pltpu.SemaphoreType.DMA,
        ],
    )
    def k(src_hbm, dst_hbm, ee_hbm, da_hbm, db_hbm, h_hbm, z_hbm,
          oa_hbm, ob_hbm, idx_s, idx_d, eb0, eb1, dn0, dn1, albuf,
          hr0, hr1, ms0, ms1, accbuf, osh, dsum,
          sh0, sh1, sd0, sd1, se0, se1, sw0, sw1):
        c = lax.axis_index("c")
        s = lax.axis_index("s")
        gid0 = jnp.where(c == 0, s * N0C2, NS * N0C2 + s * N1C2)
        nch = jnp.where(c == 0, N0C2, N1C2)
        start = jnp.minimum(gid0, TCHN2 - NMXC2)
        off = gid0 - start
        pltpu.sync_copy(src_hbm.at[pl.ds(start, NMXC2)], idx_s)
        pltpu.sync_copy(dst_hbm.at[pl.ds(start, NMXC2)], idx_d)
        pltpu.sync_copy(z_hbm.at[pl.ds(s * RS, RS)], osh.at[pl.ds(s * RS, RS)])
        for r in range(RS // CH2):
            row0 = s * RS + r * CH2
            pltpu.sync_copy(da_hbm.at[pl.ds(row0, CH2)], eb0)
            pltpu.sync_copy(db_hbm.at[pl.ds(row0, CH2)], eb1)
            for i in range(CH2):
                eb0[i, :] = eb0[i, :] + eb1[i, :] + 1e-16
            pltpu.sync_copy(eb0, dsum.at[pl.ds(row0, CH2)])
        accbuf[pl.ds(80, 16)] = jnp.zeros((16,), _f32)
        plsc.subcore_barrier()
        ebuf = [eb0, eb1]
        dnr = [dn0, dn1]
        hrw = [hr0, hr1]
        msg = [ms0, ms1]
        semh = [sh0, sh1]
        semd = [sd0, sd1]
        seme = [se0, se1]
        semw = [sw0, sw1]
        iomask = lax.iota(_i32, 16) < 8

        def issue(g, b):
            pltpu.async_copy(h_hbm.at[idx_s.at[off + g]], hrw[b], semh[b])
            pltpu.async_copy(dsum.at[idx_d.at[off + g]], dnr[b], semd[b])
            pltpu.async_copy(ee_hbm.at[pl.ds((gid0 + g) * CH2, CH2)],
                             ebuf[b], seme[b])

        issue(0, 0)
        issue(1, 1)

        @pl.loop(0, nch // 2)
        def _gg(gg):
            for b in range(2):
                g = gg * 2 + b
                pltpu.make_async_copy(h_hbm.at[idx_s.at[off + g]], hrw[b],
                                      semh[b]).wait()
                pltpu.make_async_copy(dsum.at[idx_d.at[off + g]], dnr[b],
                                      semd[b]).wait()
                pltpu.make_async_copy(ee_hbm.at[pl.ds((gid0 + g) * CH2, CH2)],
                                      ebuf[b], seme[b]).wait()

                @pl.when(gg >= 1)
                def _():
                    pltpu.make_async_copy(
                        msg[b], osh.at[idx_d.at[off + g]], semw[b]).wait()

                for i in range(CH2):
                    albuf[pl.ds(i * W16, W16)] = ebuf[b][i, :] / dnr[b][i, :]

                @pl.loop(0, CH2)
                def _edge(j):
                    av = albuf[pl.ds(j * W16, W16)]
                    acc = [jnp.zeros((16,), _f32) for _ in range(5)]
                    for p in range(4):
                        off = p * 80
                        me = jnp.full((16,), av[2 * p], _f32)
                        mo = jnp.full((16,), av[2 * p + 1], _f32)
                        mm = jnp.where(iomask, me, mo)
                        acc[0] = acc[0] + me * hrw[b][j, pl.ds(off, 16)]
                        acc[1] = acc[1] + me * hrw[b][j, pl.ds(off + 16, 16)]
                        acc[2] = acc[2] + mm * hrw[b][j, pl.ds(off + 32, 16)]
                        acc[3] = acc[3] + mo * hrw[b][j, pl.ds(off + 48, 16)]
                        acc[4] = acc[4] + mo * hrw[b][j, pl.ds(off + 64, 16)]
                    for q in range(5):
                        accbuf[pl.ds(q * 16, 16)] = acc[q]
                    msg[b][j, pl.ds(0, 16)] = (accbuf[pl.ds(0, 16)]
                                               + accbuf[pl.ds(40, 16)])
                    msg[b][j, pl.ds(16, 16)] = (accbuf[pl.ds(16, 16)]
                                                + accbuf[pl.ds(56, 16)])
                    msg[b][j, pl.ds(32, 16)] = (accbuf[pl.ds(32, 16)]
                                                + accbuf[pl.ds(72, 16)])

                pltpu.async_copy(msg[b], osh.at[idx_d.at[off + g]], semw[b],
                                 add=True)

                @pl.when(g + 2 < nch)
                def _():
                    issue(g + 2, b)

        for b in range(2):
            g_last = nch - 2 + b
            pltpu.make_async_copy(
                msg[b], osh.at[idx_d.at[off + g_last]], semw[b]).wait()
        plsc.subcore_barrier()

        @pl.when(c == 0)
        def _():
            pltpu.sync_copy(osh.at[pl.ds(s * RS, RS)],
                            oa_hbm.at[pl.ds(s * RS, RS)])

        @pl.when(c == 1)
        def _():
            pltpu.sync_copy(osh.at[pl.ds(s * RS, RS)],
                            ob_hbm.at[pl.ds(s * RS, RS)])

    return k(src, dst, ee, da, db, h2, z48)


_DOT = dict(preferred_element_type=_f32, precision=lax.Precision.HIGHEST)


def _tc1_body(x_ref, w_ref, as_ref, ad_ref, h_ref, s_ref, d_ref):
    h = jnp.dot(x_ref[...], w_ref[...], **_DOT)
    h_ref[...] = h
    a_s = jnp.dot(h, as_ref[...], **_DOT)
    a_d = jnp.dot(h, ad_ref[...], **_DOT)
    s_ref[...] = jnp.concatenate([a_s, a_d], axis=1)
    d_ref[...] = jnp.concatenate([a_d, a_s], axis=1)


def _tc1(x, w1, a_s, a_d):
    bn = 1024
    return pl.pallas_call(
        _tc1_body,
        grid=(NP // bn,),
        in_specs=[
            pl.BlockSpec((bn, D_IN), lambda i: (i, 0)),
            pl.BlockSpec((D_IN, F1), lambda i: (0, 0)),
            pl.BlockSpec((F1, HEADS), lambda i: (0, 0)),
            pl.BlockSpec((F1, HEADS), lambda i: (0, 0)),
        ],
        out_specs=[
            pl.BlockSpec((bn, F1), lambda i: (i, 0)),
            pl.BlockSpec((bn, W16), lambda i: (i, 0)),
            pl.BlockSpec((bn, W16), lambda i: (i, 0)),
        ],
        out_shape=[
            jax.ShapeDtypeStruct((NP, F1), _f32),
            jax.ShapeDtypeStruct((NP, W16), _f32),
            jax.ShapeDtypeStruct((NP, W16), _f32),
        ],
    )(x, w1, a_s, a_d)


def _tc2_body(oa_ref, ob_ref, b_ref, w_ref, as_ref, ad_ref,
              h_ref, s_ref, d_ref):
    t = oa_ref[...] + ob_ref[...] + b_ref[...]
    t = jnp.where(t > 0, t, jnp.exp(jnp.minimum(t, 0.0)) - 1.0)
    h = jnp.dot(t, w_ref[...], **_DOT)
    h_ref[...] = h
    a_s = jnp.dot(h, as_ref[...], **_DOT)
    a_d = jnp.dot(h, ad_ref[...], **_DOT)
    s_ref[...] = jnp.concatenate([a_s, a_d], axis=1)
    d_ref[...] = jnp.concatenate([a_d, a_s], axis=1)


def _tc2(oa, ob, b1, w2, a_s, a_d):
    bn = 1024
    return pl.pallas_call(
        _tc2_body,
        grid=(NP // bn,),
        in_specs=[
            pl.BlockSpec((bn, F1), lambda i: (i, 0)),
            pl.BlockSpec((bn, F1), lambda i: (i, 0)),
            pl.BlockSpec((1, F1), lambda i: (0, 0)),
            pl.BlockSpec((F1, F2), lambda i: (0, 0)),
            pl.BlockSpec((F2, HEADS), lambda i: (0, 0)),
            pl.BlockSpec((F2, HEADS), lambda i: (0, 0)),
        ],
        out_specs=[
            pl.BlockSpec((bn, F2), lambda i: (i, 0)),
            pl.BlockSpec((bn, W16), lambda i: (i, 0)),
            pl.BlockSpec((bn, W16), lambda i: (i, 0)),
        ],
        out_shape=[
            jax.ShapeDtypeStruct((NP, F2), _f32),
            jax.ShapeDtypeStruct((NP, W16), _f32),
            jax.ShapeDtypeStruct((NP, W16), _f32),
        ],
    )(oa, ob, b1, w2, a_s, a_d)


def _tc3_body(oa_ref, ob_ref, b_ref, o_ref):
    t = oa_ref[...] + ob_ref[...]
    o_ref[...] = t[:, :NUM_CLASSES] * (1.0 / OUT_HEADS) + b_ref[...]


def _tc3(oa, ob, b2):
    bn = 1000
    return pl.pallas_call(
        _tc3_body,
        grid=(N // bn,),
        in_specs=[
            pl.BlockSpec((bn, C_PAD), lambda i: (i, 0)),
            pl.BlockSpec((bn, C_PAD), lambda i: (i, 0)),
            pl.BlockSpec((1, NUM_CLASSES), lambda i: (0, 0)),
        ],
        out_specs=pl.BlockSpec((bn, NUM_CLASSES), lambda i: (i, 0)),
        out_shape=jax.ShapeDtypeStruct((N, NUM_CLASSES), _f32),
    )(oa, ob, b2)


def _expand_att(a, heads, ch):
    return jnp.repeat(jnp.eye(heads, dtype=_f32), ch, axis=0) * a.reshape(-1, 1)


def kernel(x, edge_index, W1, a_src1, a_dst1, b1, W2, a_src2, a_dst2, b2):
    src = edge_index[0].astype(_i32)
    dst = edge_index[1].astype(_i32)
    pad = jnp.full((EP - E,), N, _i32)
    src = jnp.concatenate([src, pad])
    dst = jnp.concatenate([dst, pad])
    src128 = src.reshape(EP // CH, CH)
    dst128 = dst.reshape(EP // CH, CH)
    src64 = src.reshape(EP // CH2, CH2)
    dst64 = dst.reshape(EP // CH2, CH2)

    xp = jnp.pad(x, ((0, NP - N), (0, 0)))
    as1 = _expand_att(a_src1, HEADS, HID)
    ad1 = _expand_att(a_dst1, HEADS, HID)
    as2 = _expand_att(a_src2, OUT_HEADS, NUM_CLASSES)
    ad2 = _expand_att(a_dst2, OUT_HEADS, NUM_CLASSES)

    z16 = jnp.zeros((NP, W16), _f32)
    z48 = jnp.zeros((NP, C_PAD), _f32)
    z64 = jnp.zeros((NP, F1), _f32)

    h1, ts1, td1 = _tc1(xp, W1, as1, ad1)
    ee1, da1, db1 = _edge_pass1(src128, dst128, ts1, td1, z16)
    oa1, ob1 = _edge_pass2_l1(src128, dst128, ee1, da1, db1, h1, z64)
    h2, ts2, td2 = _tc2(oa1, ob1, b1.reshape(1, F1), W2, as2, ad2)
    ee2, da2, db2 = _edge_pass1(src128, dst128, ts2, td2, z16)
    oa2, ob2 = _edge_pass2_l2(src64, dst64, ee2, da2, db2, h2, z48)
    return _tc3(oa2, ob2, b2.reshape(1, NUM_CLASSES))


# speedup vs baseline: 1.0638x; 1.0416x over previous
import functools

import jax
import jax.numpy as jnp
from jax import lax
from jax.experimental import pallas as pl
from jax.experimental.pallas import tpu as pltpu
from jax.experimental.pallas import tpu_sc as plsc

N = 10000
NP = 10240
E = 320000
EP = 327680
D_IN = 128
HEADS = 8
HID = 8
F1 = HEADS * HID
NUM_CLASSES = 40
OUT_HEADS = 8
F2 = OUT_HEADS * NUM_CLASSES
C_PAD = 48
W16 = 16

NC = 2
NS = 16
NT = NC * NS
EPT = EP // NT
CH = 128
NCHUNK = EPT // CH
CH2 = 64
NCHUNK2 = EPT // CH2
RS = NP // NS

N0C, N1C = 100, 60
NMXC = max(N0C, N1C)
TCHN = EP // CH
N0L1, N1L1 = 100, 60
N0C2, N1C2 = 200, 120
NMXC2 = max(N0C2, N1C2)
TCHN2 = EP // CH2

_MESH = dict(core_axis_name="c", subcore_axis_name="s", num_cores=NC,
             num_subcores=NS)

_f32 = jnp.float32
_i32 = jnp.int32


def _edge_pass1(src, dst, ts, td, z16):
    mesh = plsc.VectorSubcoreMesh(**_MESH)

    @functools.partial(
        pl.kernel,
        out_type=[
            jax.ShapeDtypeStruct((EP, W16), _f32),
            jax.ShapeDtypeStruct((NP, W16), _f32),
            jax.ShapeDtypeStruct((NP, W16), _f32),
        ],
        mesh=mesh,
        compiler_params=pltpu.CompilerParams(use_tc_tiling_on_sc=False),
        scratch_types=[
            pltpu.VMEM((NMXC, CH), _i32),
            pltpu.VMEM((NMXC, CH), _i32),
            pltpu.VMEM((CH, W16), _f32),
            pltpu.VMEM((CH, W16), _f32),
            pltpu.VMEM((CH, W16), _f32),
            pltpu.VMEM((CH, W16), _f32),
            pltpu.VMEM((CH, W16), _f32),
            pltpu.VMEM((CH, W16), _f32),
            pltpu.VMEM_SHARED((NP, W16), _f32),
            pltpu.SemaphoreType.DMA,
            pltpu.SemaphoreType.DMA,
            pltpu.SemaphoreType.DMA,
            pltpu.SemaphoreType.DMA,
            pltpu.SemaphoreType.DMA,
            pltpu.SemaphoreType.DMA,
            pltpu.SemaphoreType.DMA,
            pltpu.SemaphoreType.DMA,
        ],
    )
    def k(src_hbm, dst_hbm, ts_hbm, td_hbm, z_hbm, ee_hbm, da_hbm, db_hbm,
          idx_s, idx_d, sr0, sr1, dr0, dr1, eb0, eb1, dsh,
          ss0, ss1, sd0, sd1, se0, se1, sw0, sw1):
        c = lax.axis_index("c")
        s = lax.axis_index("s")
        gid0 = jnp.where(c == 0, s * N0C, NS * N0C + s * N1C)
        nch = jnp.where(c == 0, N0C, N1C)
        start = jnp.minimum(gid0, TCHN - NMXC)
        off = gid0 - start
        pltpu.sync_copy(src_hbm.at[pl.ds(start, NMXC)], idx_s)
        pltpu.sync_copy(dst_hbm.at[pl.ds(start, NMXC)], idx_d)
        pltpu.sync_copy(z_hbm.at[pl.ds(s * RS, RS)], dsh.at[pl.ds(s * RS, RS)])
        plsc.subcore_barrier()
        srows = [sr0, sr1]
        drows = [dr0, dr1]
        ebuf = [eb0, eb1]
        sems = [ss0, ss1]
        semd = [sd0, sd1]
        seme = [se0, se1]
        semw = [sw0, sw1]

        def issue(g, b):
            pltpu.async_copy(ts_hbm.at[idx_s.at[off + g]], srows[b], sems[b])
            pltpu.async_copy(td_hbm.at[idx_d.at[off + g]], drows[b], semd[b])

        issue(0, 0)
        issue(1, 1)

        @pl.loop(0, nch // 2)
        def _gg(gg):
            for b in range(2):
                g = gg * 2 + b
                pltpu.make_async_copy(ts_hbm.at[idx_s.at[off + g]], srows[b],
                                      sems[b]).wait()
                pltpu.make_async_copy(td_hbm.at[idx_d.at[off + g]], drows[b],
                                      semd[b]).wait()

                @pl.when(gg >= 1)
                def _():
                    pltpu.make_async_copy(
                        ebuf[b], ee_hbm.at[pl.ds((gid0 + g) * CH, CH)],
                        seme[b]).wait()
                    pltpu.make_async_copy(
                        ebuf[b], dsh.at[idx_d.at[off + g]], semw[b]).wait()

                for i in range(CH):
                    v = srows[b][i, :] + drows[b][i, :]
                    v = jnp.maximum(v, 0.2 * v)
                    ebuf[b][i, :] = jnp.exp(v)
                pltpu.async_copy(
                    ebuf[b], ee_hbm.at[pl.ds((gid0 + g) * CH, CH)], seme[b])
                pltpu.async_copy(
                    ebuf[b], dsh.at[idx_d.at[off + g]], semw[b], add=True)

                @pl.when(g + 2 < nch)
                def _():
                    issue(g + 2, b)

        for b in range(2):
            g_last = nch - 2 + b
            pltpu.make_async_copy(
                ebuf[b], ee_hbm.at[pl.ds((gid0 + g_last) * CH, CH)],
                seme[b]).wait()
            pltpu.make_async_copy(
                ebuf[b], dsh.at[idx_d.at[off + g_last]], semw[b]).wait()
        plsc.subcore_barrier()

        @pl.when(c == 0)
        def _():
            pltpu.sync_copy(dsh.at[pl.ds(s * RS, RS)],
                            da_hbm.at[pl.ds(s * RS, RS)])

        @pl.when(c == 1)
        def _():
            pltpu.sync_copy(dsh.at[pl.ds(s * RS, RS)],
                            db_hbm.at[pl.ds(s * RS, RS)])

    return k(src, dst, ts, td, z16)


def _edge_pass2_l1(src, dst, ee, da, db, h, z64):
    mesh = plsc.VectorSubcoreMesh(**_MESH)

    @functools.partial(
        pl.kernel,
        out_type=[
            jax.ShapeDtypeStruct((NP, F1), _f32),
            jax.ShapeDtypeStruct((NP, F1), _f32),
        ],
        mesh=mesh,
        compiler_params=pltpu.CompilerParams(use_tc_tiling_on_sc=False),
        scratch_types=[
            pltpu.VMEM((NMXC, CH), _i32),
            pltpu.VMEM((NMXC, CH), _i32),
            pltpu.VMEM((CH, W16), _f32),
            pltpu.VMEM((CH, W16), _f32),
            pltpu.VMEM((CH, W16), _f32),
            pltpu.VMEM((CH, W16), _f32),
            pltpu.VMEM((CH * W16,), _f32),
            pltpu.VMEM((CH, F1), _f32),
            pltpu.VMEM((CH, F1), _f32),
            pltpu.VMEM((CH, F1), _f32),
            pltpu.VMEM((CH, F1), _f32),
            pltpu.VMEM_SHARED((NP, F1), _f32),
            pltpu.VMEM_SHARED((NP, W16), _f32),
            pltpu.SemaphoreType.DMA,
            pltpu.SemaphoreType.DMA,
            pltpu.SemaphoreType.DMA,
            pltpu.SemaphoreType.DMA,
            pltpu.SemaphoreType.DMA,
            pltpu.SemaphoreType.DMA,
            pltpu.SemaphoreType.DMA,
            pltpu.SemaphoreType.DMA,
        ],
    )
    def k(src_hbm, dst_hbm, ee_hbm, da_hbm, db_hbm, h_hbm, z_hbm,
          oa_hbm, ob_hbm, idx_s, idx_d, eb0, eb1, dn0, dn1, albuf,
          hr0, hr1, ms0, ms1, osh, dsum,
          sh0, sh1, sd0, sd1, se0, se1, sw0, sw1):
        c = lax.axis_index("c")
        s = lax.axis_index("s")
        gid0 = jnp.where(c == 0, s * N0L1, NS * N0L1 + s * N1L1)
        nch = jnp.where(c == 0, N0L1, N1L1)
        start = jnp.minimum(gid0, TCHN - NMXC)
        off = gid0 - start
        pltpu.sync_copy(src_hbm.at[pl.ds(start, NMXC)], idx_s)
        pltpu.sync_copy(dst_hbm.at[pl.ds(start, NMXC)], idx_d)
        pltpu.sync_copy(z_hbm.at[pl.ds(s * RS, RS)], osh.at[pl.ds(s * RS, RS)])
        for r in range(RS // CH):
            row0 = s * RS + r * CH
            pltpu.sync_copy(da_hbm.at[pl.ds(row0, CH)], eb0)
            pltpu.sync_copy(db_hbm.at[pl.ds(row0, CH)], eb1)
            for i in range(CH):
                eb0[i, :] = eb0[i, :] + eb1[i, :] + 1e-16
            pltpu.sync_copy(eb0, dsum.at[pl.ds(row0, CH)])
        plsc.subcore_barrier()
        ebuf = [eb0, eb1]
        dnr = [dn0, dn1]
        hrw = [hr0, hr1]
        msg = [ms0, ms1]
        semh = [sh0, sh1]
        semd = [sd0, sd1]
        seme = [se0, se1]
        semw = [sw0, sw1]
        iomask = lax.iota(_i32, 16) < 8

        def issue(g, b):
            pltpu.async_copy(h_hbm.at[idx_s.at[off + g]], hrw[b], semh[b])
            pltpu.async_copy(dsum.at[idx_d.at[off + g]], dnr[b], semd[b])
            pltpu.async_copy(ee_hbm.at[pl.ds((gid0 + g) * CH, CH)],
                             ebuf[b], seme[b])

        issue(0, 0)
        issue(1, 1)

        @pl.loop(0, nch // 2)
        def _gg(gg):
            for b in range(2):
                g = gg * 2 + b
                pltpu.make_async_copy(h_hbm.at[idx_s.at[off + g]], hrw[b],
                                      semh[b]).wait()
                pltpu.make_async_copy(dsum.at[idx_d.at[off + g]], dnr[b],
                                      semd[b]).wait()
                pltpu.make_async_copy(ee_hbm.at[pl.ds((gid0 + g) * CH, CH)],
                                      ebuf[b], seme[b]).wait()

                @pl.when(gg >= 1)
                def _():
                    pltpu.make_async_copy(
                        msg[b], osh.at[idx_d.at[off + g]], semw[b]).wait()

                for i in range(CH):
                    albuf[pl.ds(i * W16, W16)] = ebuf[b][i, :] / dnr[b][i, :]

                @pl.loop(0, CH)
                def _edge(j):
                    av = albuf[pl.ds(j * W16, W16)]
                    for kq in range(4):
                        v = hrw[b][j, pl.ds(kq * 16, 16)]
                        me = jnp.full((16,), av[2 * kq], _f32)
                        mo = jnp.full((16,), av[2 * kq + 1], _f32)
                        msg[b][j, pl.ds(kq * 16, 16)] = (
                            v * jnp.where(iomask, me, mo))

                pltpu.async_copy(msg[b], osh.at[idx_d.at[off + g]], semw[b],
                                 add=True)

                @pl.when(g + 2 < nch)
                def _():
                    issue(g + 2, b)

        for b in range(2):
            g_last = nch - 2 + b
            pltpu.make_async_copy(
                msg[b], osh.at[idx_d.at[off + g_last]], semw[b]).wait()
        plsc.subcore_barrier()

        @pl.when(c == 0)
        def _():
            pltpu.sync_copy(osh.at[pl.ds(s * RS, RS)],
                            oa_hbm.at[pl.ds(s * RS, RS)])

        @pl.when(c == 1)
        def _():
            pltpu.sync_copy(osh.at[pl.ds(s * RS, RS)],
                            ob_hbm.at[pl.ds(s * RS, RS)])

    return k(src, dst, ee, da, db, h, z64)


def _edge_pass2_l2(src, dst, ee, da, db, h2, z48):
    mesh = plsc.VectorSubcoreMesh(**_MESH)

    @functools.partial(
        pl.kernel,
        out_type=[
            jax.ShapeDtypeStruct((NP, C_PAD), _f32),
            jax.ShapeDtypeStruct((NP, C_PAD), _f32),
        ],
        mesh=mesh,
        compiler_params=pltpu.CompilerParams(use_tc_tiling_on_sc=False),
        scratch_types=[
            pltpu.VMEM((NMXC2, CH2), _i32),
            pltpu.VMEM((NMXC2, CH2), _i32),
            pltpu.VMEM((CH2, W16), _f32),
            pltpu.VMEM((CH2, W16), _f32),
            pltpu.VMEM((CH2, W16), _f32),
            pltpu.VMEM((CH2, W16), _f32),
            pltpu.VMEM((CH2 * W16,), _f32),
            pltpu.VMEM((CH2, F2), _f32),
            pltpu.VMEM((CH2, F2), _f32),
            pltpu.VMEM((CH2, C_PAD), _f32),
            pltpu.VMEM((CH2, C_PAD), _f32),
            pltpu.VMEM((96,), _f32),
            pltpu.VMEM_SHARED((NP, C_PAD), _f32),
            pltpu.VMEM_SHARED((NP, W16), _f32),
            pltpu.SemaphoreType.DMA,
            pltpu.SemaphoreType.DMA,
            pltpu.SemaphoreType.DMA,
            pltpu.SemaphoreType.DMA,
            pltpu.SemaphoreType.DMA,
            pltpu.SemaphoreType.DMA,
            pltpu.SemaphoreType.DMA,
            pltpu.SemaphoreType.DMA,
        ],
    )
    def k(src_hbm, dst_hbm, ee_hbm, da_hbm, db_hbm, h_hbm, z_hbm,
          oa_hbm, ob_hbm, idx_s, idx_d, eb0, eb1, dn0, dn1, albuf,
          hr0, hr1, ms0, ms1, accbuf, osh, dsum,
          sh0, sh1, sd0, sd1, se0, se1, sw0, sw1):
        c = lax.axis_index("c")
        s = lax.axis_index("s")
        gid0 = jnp.where(c == 0, s * N0C2, NS * N0C2 + s * N1C2)
        nch = jnp.where(c == 0, N0C2, N1C2)
        start = jnp.minimum(gid0, TCHN2 - NMXC2)
        off = gid0 - start
        pltpu.sync_copy(src_hbm.at[pl.ds(start, NMXC2)], idx_s)
        pltpu.sync_copy(dst_hbm.at[pl.ds(start, NMXC2)], idx_d)
        pltpu.sync_copy(z_hbm.at[pl.ds(s * RS, RS)], osh.at[pl.ds(s * RS, RS)])
        for r in range(RS // CH2):
            row0 = s * RS + r * CH2
            pltpu.sync_copy(da_hbm.at[pl.ds(row0, CH2)], eb0)
            pltpu.sync_copy(db_hbm.at[pl.ds(row0, CH2)], eb1)
            for i in range(CH2):
                eb0[i, :] = eb0[i, :] + eb1[i, :] + 1e-16
            pltpu.sync_copy(eb0, dsum.at[pl.ds(row0, CH2)])
        accbuf[pl.ds(80, 16)] = jnp.zeros((16,), _f32)
        plsc.subcore_barrier()
        ebuf = [eb0, eb1]
        dnr = [dn0, dn1]
        hrw = [hr0, hr1]
        msg = [ms0, ms1]
        semh = [sh0, sh1]
        semd = [sd0, sd1]
        seme = [se0, se1]
        semw = [sw0, sw1]
        iomask = lax.iota(_i32, 16) < 8

        def issue(g, b):
            pltpu.async_copy(h_hbm.at[idx_s.at[off + g]], hrw[b], semh[b])
            pltpu.async_copy(dsum.at[idx_d.at[off + g]], dnr[b], semd[b])
            pltpu.async_copy(ee_hbm.at[pl.ds((gid0 + g) * CH2, CH2)],
                             ebuf[b], seme[b])

        issue(0, 0)
        issue(1, 1)

        @pl.loop(0, nch // 2)
        def _gg(gg):
            for b in range(2):
                g = gg * 2 + b
                pltpu.make_async_copy(h_hbm.at[idx_s.at[off + g]], hrw[b],
                                      semh[b]).wait()
                pltpu.make_async_copy(dsum.at[idx_d.at[off + g]], dnr[b],
                                      semd[b]).wait()
                pltpu.make_async_copy(ee_hbm.at[pl.ds((gid0 + g) * CH2, CH2)],
                                      ebuf[b], seme[b]).wait()

                @pl.when(gg >= 1)
                def _():
                    pltpu.make_async_copy(
                        msg[b], osh.at[idx_d.at[off + g]], semw[b]).wait()

                for i in range(CH2):
                    albuf[pl.ds(i * W16, W16)] = ebuf[b][i, :] / dnr[b][i, :]

                @pl.loop(0, CH2)
                def _edge(j):
                    av = albuf[pl.ds(j * W16, W16)]
                    acc = [jnp.zeros((16,), _f32) for _ in range(5)]
                    for p in range(4):
                        off = p * 80
                        me = jnp.full((16,), av[2 * p], _f32)
                        mo = jnp.full((16,), av[2 * p + 1], _f32)
                        mm = jnp.where(iomask, me, mo)
                        acc[0] = acc[0] + me * hrw[b][j, pl.ds(off, 16)]
                        acc[1] = acc[1] + me * hrw[b][j, pl.ds(off + 16, 16)]
                        acc[2] = acc[2] + mm * hrw[b][j, pl.ds(off + 32, 16)]
                        acc[3] = acc[3] + mo * hrw[b][j, pl.ds(off + 48, 16)]
                        acc[4] = acc[4] + mo * hrw[b][j, pl.ds(off + 64, 16)]
                    for q in range(5):
                        accbuf[pl.ds(q * 16, 16)] = acc[q]
                    msg[b][j, pl.ds(0, 16)] = (accbuf[pl.ds(0, 16)]
                                               + accbuf[pl.ds(40, 16)])
                    msg[b][j, pl.ds(16, 16)] = (accbuf[pl.ds(16, 16)]
                                                + accbuf[pl.ds(56, 16)])
                    msg[b][j, pl.ds(32, 16)] = (accbuf[pl.ds(32, 16)]
                                                + accbuf[pl.ds(72, 16)])

                pltpu.async_copy(msg[b], osh.at[idx_d.at[off + g]], semw[b],
                                 add=True)

                @pl.when(g + 2 < nch)
                def _():
                    issue(g + 2, b)

        for b in range(2):
            g_last = nch - 2 + b
            pltpu.make_async_copy(
                msg[b], osh.at[idx_d.at[off + g_last]], semw[b]).wait()
        plsc.subcore_barrier()

        @pl.when(c == 0)
        def _():
            pltpu.sync_copy(osh.at[pl.ds(s * RS, RS)],
                            oa_hbm.at[pl.ds(s * RS, RS)])

        @pl.when(c == 1)
        def _():
            pltpu.sync_copy(osh.at[pl.ds(s * RS, RS)],
                            ob_hbm.at[pl.ds(s * RS, RS)])

    return k(src, dst, ee, da, db, h2, z48)


_DOT = dict(preferred_element_type=_f32, precision=lax.Precision.HIGHEST)


def _tc1_body(x_ref, w_ref, as_ref, ad_ref, h_ref, s_ref, d_ref):
    h = jnp.dot(x_ref[...], w_ref[...], **_DOT)
    h_ref[...] = h
    a_s = jnp.dot(h, as_ref[...], **_DOT)
    a_d = jnp.dot(h, ad_ref[...], **_DOT)
    s_ref[...] = jnp.concatenate([a_s, a_d], axis=1)
    d_ref[...] = jnp.concatenate([a_d, a_s], axis=1)


def _tc1(x, w1, a_s, a_d):
    bn = 1024
    return pl.pallas_call(
        _tc1_body,
        grid=(NP // bn,),
        in_specs=[
            pl.BlockSpec((bn, D_IN), lambda i: (i, 0)),
            pl.BlockSpec((D_IN, F1), lambda i: (0, 0)),
            pl.BlockSpec((F1, HEADS), lambda i: (0, 0)),
            pl.BlockSpec((F1, HEADS), lambda i: (0, 0)),
        ],
        out_specs=[
            pl.BlockSpec((bn, F1), lambda i: (i, 0)),
            pl.BlockSpec((bn, W16), lambda i: (i, 0)),
            pl.BlockSpec((bn, W16), lambda i: (i, 0)),
        ],
        out_shape=[
            jax.ShapeDtypeStruct((NP, F1), _f32),
            jax.ShapeDtypeStruct((NP, W16), _f32),
            jax.ShapeDtypeStruct((NP, W16), _f32),
        ],
    )(x, w1, a_s, a_d)


def _tc2_body(oa_ref, ob_ref, b_ref, w_ref, as_ref, ad_ref,
              h_ref, s_ref, d_ref):
    t = oa_ref[...] + ob_ref[...] + b_ref[...]
    t = jnp.where(t > 0, t, jnp.exp(jnp.minimum(t, 0.0)) - 1.0)
    h = jnp.dot(t, w_ref[...], **_DOT)
    h_ref[...] = h
    a_s = jnp.dot(h, as_ref[...], **_DOT)
    a_d = jnp.dot(h, ad_ref[...], **_DOT)
    s_ref[...] = jnp.concatenate([a_s, a_d], axis=1)
    d_ref[...] = jnp.concatenate([a_d, a_s], axis=1)


def _tc2(oa, ob, b1, w2, a_s, a_d):
    bn = 1024
    return pl.pallas_call(
        _tc2_body,
        grid=(NP // bn,),
        in_specs=[
            pl.BlockSpec((bn, F1), lambda i: (i, 0)),
            pl.BlockSpec((bn, F1), lambda i: (i, 0)),
            pl.BlockSpec((1, F1), lambda i: (0, 0)),
            pl.BlockSpec((F1, F2), lambda i: (0, 0)),
            pl.BlockSpec((F2, HEADS), lambda i: (0, 0)),
            pl.BlockSpec((F2, HEADS), lambda i: (0, 0)),
        ],
        out_specs=[
            pl.BlockSpec((bn, F2), lambda i: (i, 0)),
            pl.BlockSpec((bn, W16), lambda i: (i, 0)),
            pl.BlockSpec((bn, W16), lambda i: (i, 0)),
        ],
        out_shape=[
            jax.ShapeDtypeStruct((NP, F2), _f32),
            jax.ShapeDtypeStruct((NP, W16), _f32),
            jax.ShapeDtypeStruct((NP, W16), _f32),
        ],
    )(oa, ob, b1, w2, a_s, a_d)


def _tc3_body(oa_ref, ob_ref, b_ref, o_ref):
    t = oa_ref[...] + ob_ref[...]
    o_ref[...] = t[:, :NUM_CLASSES] * (1.0 / OUT_HEADS) + b_ref[...]


def _tc3(oa, ob, b2):
    bn = 1000
    return pl.pallas_call(
        _tc3_body,
        grid=(N // bn,),
        in_specs=[
            pl.BlockSpec((bn, C_PAD), lambda i: (i, 0)),
            pl.BlockSpec((bn, C_PAD), lambda i: (i, 0)),
            pl.BlockSpec((1, NUM_CLASSES), lambda i: (0, 0)),
        ],
        out_specs=pl.BlockSpec((bn, NUM_CLASSES), lambda i: (i, 0)),
        out_shape=jax.ShapeDtypeStruct((N, NUM_CLASSES), _f32),
    )(oa, ob, b2)


def _expand_att(a, heads, ch):
    return jnp.repeat(jnp.eye(heads, dtype=_f32), ch, axis=0) * a.reshape(-1, 1)


def kernel(x, edge_index, W1, a_src1, a_dst1, b1, W2, a_src2, a_dst2, b2):
    src = edge_index[0].astype(_i32)
    dst = edge_index[1].astype(_i32)
    pad = jnp.full((EP - E,), N, _i32)
    src = jnp.concatenate([src, pad])
    dst = jnp.concatenate([dst, pad])
    src128 = src.reshape(EP // CH, CH)
    dst128 = dst.reshape(EP // CH, CH)
    src64 = src.reshape(EP // CH2, CH2)
    dst64 = dst.reshape(EP // CH2, CH2)

    xp = jnp.pad(x, ((0, NP - N), (0, 0)))
    as1 = _expand_att(a_src1, HEADS, HID)
    ad1 = _expand_att(a_dst1, HEADS, HID)
    as2 = _expand_att(a_src2, OUT_HEADS, NUM_CLASSES)
    ad2 = _expand_att(a_dst2, OUT_HEADS, NUM_CLASSES)

    z16 = jnp.zeros((NP, W16), _f32)
    z48 = jnp.zeros((NP, C_PAD), _f32)
    z64 = jnp.zeros((NP, F1), _f32)

    h1, ts1, td1 = _tc1(xp, W1, as1, ad1)
    ee1, da1, db1 = _edge_pass1(src128, dst128, ts1, td1, z16)
    oa1, ob1 = _edge_pass2_l1(src128, dst128, ee1, da1, db1, h1, z64)
    h2, ts2, td2 = _tc2(oa1, ob1, b1.reshape(1, F1), W2, as2, ad2)
    ee2, da2, db2 = _edge_pass1(src128, dst128, ts2, td2, z16)
    oa2, ob2 = _edge_pass2_l2(src64, dst64, ee2, da2, db2, h2, z48)
    return _tc3(oa2, ob2, b2.reshape(1, NUM_CLASSES))


# speedup vs baseline: 1.0785x; 1.0139x over previous
import functools

import jax
import jax.numpy as jnp
from jax import lax
from jax.experimental import pallas as pl
from jax.experimental.pallas import tpu as pltpu
from jax.experimental.pallas import tpu_sc as plsc

N = 10000
NP = 10240
E = 320000
EP = 327680
D_IN = 128
HEADS = 8
HID = 8
F1 = HEADS * HID
NUM_CLASSES = 40
OUT_HEADS = 8
F2 = OUT_HEADS * NUM_CLASSES
C_PAD = 48
W16 = 16

NC = 2
NS = 16
NT = NC * NS
EPT = EP // NT
CH = 128
NCHUNK = EPT // CH
CH2 = 64
NCHUNK2 = EPT // CH2
RS = NP // NS

N0C, N1C = 108, 52
NMXC = max(N0C, N1C)
TCHN = EP // CH
N0L1, N1L1 = 108, 52
N0C2, N1C2 = 216, 104
NMXC2 = max(N0C2, N1C2)
TCHN2 = EP // CH2

_MESH = dict(core_axis_name="c", subcore_axis_name="s", num_cores=NC,
             num_subcores=NS)

_f32 = jnp.float32
_i32 = jnp.int32


def _edge_pass1(src, dst, ts, td, z16):
    mesh = plsc.VectorSubcoreMesh(**_MESH)

    @functools.partial(
        pl.kernel,
        out_type=[
            jax.ShapeDtypeStruct((EP, W16), _f32),
            jax.ShapeDtypeStruct((NP, W16), _f32),
            jax.ShapeDtypeStruct((NP, W16), _f32),
        ],
        mesh=mesh,
        compiler_params=pltpu.CompilerParams(use_tc_tiling_on_sc=False),
        scratch_types=[
            pltpu.VMEM((NMXC, CH), _i32),
            pltpu.VMEM((NMXC, CH), _i32),
            pltpu.VMEM((CH, W16), _f32),
            pltpu.VMEM((CH, W16), _f32),
            pltpu.VMEM((CH, W16), _f32),
            pltpu.VMEM((CH, W16), _f32),
            pltpu.VMEM((CH, W16), _f32),
            pltpu.VMEM((CH, W16), _f32),
            pltpu.VMEM_SHARED((NP, W16), _f32),
            pltpu.SemaphoreType.DMA,
            pltpu.SemaphoreType.DMA,
            pltpu.SemaphoreType.DMA,
            pltpu.SemaphoreType.DMA,
            pltpu.SemaphoreType.DMA,
            pltpu.SemaphoreType.DMA,
            pltpu.SemaphoreType.DMA,
            pltpu.SemaphoreType.DMA,
        ],
    )
    def k(src_hbm, dst_hbm, ts_hbm, td_hbm, z_hbm, ee_hbm, da_hbm, db_hbm,
          idx_s, idx_d, sr0, sr1, dr0, dr1, eb0, eb1, dsh,
          ss0, ss1, sd0, sd1, se0, se1, sw0, sw1):
        c = lax.axis_index("c")
        s = lax.axis_index("s")
        gid0 = jnp.where(c == 0, s * N0C, NS * N0C + s * N1C)
        nch = jnp.where(c == 0, N0C, N1C)
        start = jnp.minimum(gid0, TCHN - NMXC)
        off = gid0 - start
        pltpu.sync_copy(src_hbm.at[pl.ds(start, NMXC)], idx_s)
        pltpu.sync_copy(dst_hbm.at[pl.ds(start, NMXC)], idx_d)
        pltpu.sync_copy(z_hbm.at[pl.ds(s * RS, RS)], dsh.at[pl.ds(s * RS, RS)])
        plsc.subcore_barrier()
        srows = [sr0, sr1]
        drows = [dr0, dr1]
        ebuf = [eb0, eb1]
        sems = [ss0, ss1]
        semd = [sd0, sd1]
        seme = [se0, se1]
        semw = [sw0, sw1]

        def issue(g, b):
            pltpu.async_copy(ts_hbm.at[idx_s.at[off + g]], srows[b], sems[b])
            pltpu.async_copy(td_hbm.at[idx_d.at[off + g]], drows[b], semd[b])

        issue(0, 0)
        issue(1, 1)

        @pl.loop(0, nch // 2)
        def _gg(gg):
            for b in range(2):
                g = gg * 2 + b
                pltpu.make_async_copy(ts_hbm.at[idx_s.at[off + g]], srows[b],
                                      sems[b]).wait()
                pltpu.make_async_copy(td_hbm.at[idx_d.at[off + g]], drows[b],
                                      semd[b]).wait()

                @pl.when(gg >= 1)
                def _():
                    pltpu.make_async_copy(
                        ebuf[b], ee_hbm.at[pl.ds((gid0 + g) * CH, CH)],
                        seme[b]).wait()
                    pltpu.make_async_copy(
                        ebuf[b], dsh.at[idx_d.at[off + g]], semw[b]).wait()

                for i in range(CH):
                    v = srows[b][i, :] + drows[b][i, :]
                    v = jnp.maximum(v, 0.2 * v)
                    ebuf[b][i, :] = jnp.exp(v)
                pltpu.async_copy(
                    ebuf[b], ee_hbm.at[pl.ds((gid0 + g) * CH, CH)], seme[b])
                pltpu.async_copy(
                    ebuf[b], dsh.at[idx_d.at[off + g]], semw[b], add=True)

                @pl.when(g + 2 < nch)
                def _():
                    issue(g + 2, b)

        for b in range(2):
            g_last = nch - 2 + b
            pltpu.make_async_copy(
                ebuf[b], ee_hbm.at[pl.ds((gid0 + g_last) * CH, CH)],
                seme[b]).wait()
            pltpu.make_async_copy(
                ebuf[b], dsh.at[idx_d.at[off + g_last]], semw[b]).wait()
        plsc.subcore_barrier()

        @pl.when(c == 0)
        def _():
            pltpu.sync_copy(dsh.at[pl.ds(s * RS, RS)],
                            da_hbm.at[pl.ds(s * RS, RS)])

        @pl.when(c == 1)
        def _():
            pltpu.sync_copy(dsh.at[pl.ds(s * RS, RS)],
                            db_hbm.at[pl.ds(s * RS, RS)])

    return k(src, dst, ts, td, z16)


def _edge_pass2_l1(src, dst, ee, da, db, h, z64):
    mesh = plsc.VectorSubcoreMesh(**_MESH)

    @functools.partial(
        pl.kernel,
        out_type=[
            jax.ShapeDtypeStruct((NP, F1), _f32),
            jax.ShapeDtypeStruct((NP, F1), _f32),
        ],
        mesh=mesh,
        compiler_params=pltpu.CompilerParams(use_tc_tiling_on_sc=False),
        scratch_types=[
            pltpu.VMEM((NMXC, CH), _i32),
            pltpu.VMEM((NMXC, CH), _i32),
            pltpu.VMEM((CH, W16), _f32),
            pltpu.VMEM((CH, W16), _f32),
            pltpu.VMEM((CH, W16), _f32),
            pltpu.VMEM((CH, W16), _f32),
            pltpu.VMEM((CH * W16,), _f32),
            pltpu.VMEM((CH, F1), _f32),
            pltpu.VMEM((CH, F1), _f32),
            pltpu.VMEM((CH, F1), _f32),
            pltpu.VMEM((CH, F1), _f32),
            pltpu.VMEM_SHARED((NP, F1), _f32),
            pltpu.VMEM_SHARED((NP, W16), _f32),
            pltpu.SemaphoreType.DMA,
            pltpu.SemaphoreType.DMA,
            pltpu.SemaphoreType.DMA,
            pltpu.SemaphoreType.DMA,
            pltpu.SemaphoreType.DMA,
            pltpu.SemaphoreType.DMA,
            pltpu.SemaphoreType.DMA,
            pltpu.SemaphoreType.DMA,
        ],
    )
    def k(src_hbm, dst_hbm, ee_hbm, da_hbm, db_hbm, h_hbm, z_hbm,
          oa_hbm, ob_hbm, idx_s, idx_d, eb0, eb1, dn0, dn1, albuf,
          hr0, hr1, ms0, ms1, osh, dsum,
          sh0, sh1, sd0, sd1, se0, se1, sw0, sw1):
        c = lax.axis_index("c")
        s = lax.axis_index("s")
        gid0 = jnp.where(c == 0, s * N0L1, NS * N0L1 + s * N1L1)
        nch = jnp.where(c == 0, N0L1, N1L1)
        start = jnp.minimum(gid0, TCHN - NMXC)
        off = gid0 - start
        pltpu.sync_copy(src_hbm.at[pl.ds(start, NMXC)], idx_s)
        pltpu.sync_copy(dst_hbm.at[pl.ds(start, NMXC)], idx_d)
        pltpu.sync_copy(z_hbm.at[pl.ds(s * RS, RS)], osh.at[pl.ds(s * RS, RS)])
        for r in range(RS // CH):
            row0 = s * RS + r * CH
            pltpu.sync_copy(da_hbm.at[pl.ds(row0, CH)], eb0)
            pltpu.sync_copy(db_hbm.at[pl.ds(row0, CH)], eb1)
            for i in range(CH):
                eb0[i, :] = eb0[i, :] + eb1[i, :] + 1e-16
            pltpu.sync_copy(eb0, dsum.at[pl.ds(row0, CH)])
        plsc.subcore_barrier()
        ebuf = [eb0, eb1]
        dnr = [dn0, dn1]
        hrw = [hr0, hr1]
        msg = [ms0, ms1]
        semh = [sh0, sh1]
        semd = [sd0, sd1]
        seme = [se0, se1]
        semw = [sw0, sw1]
        iomask = lax.iota(_i32, 16) < 8

        def issue(g, b):
            pltpu.async_copy(h_hbm.at[idx_s.at[off + g]], hrw[b], semh[b])
            pltpu.async_copy(dsum.at[idx_d.at[off + g]], dnr[b], semd[b])
            pltpu.async_copy(ee_hbm.at[pl.ds((gid0 + g) * CH, CH)],
                             ebuf[b], seme[b])

        issue(0, 0)
        issue(1, 1)

        @pl.loop(0, nch // 2)
        def _gg(gg):
            for b in range(2):
                g = gg * 2 + b
                pltpu.make_async_copy(h_hbm.at[idx_s.at[off + g]], hrw[b],
                                      semh[b]).wait()
                pltpu.make_async_copy(dsum.at[idx_d.at[off + g]], dnr[b],
                                      semd[b]).wait()
                pltpu.make_async_copy(ee_hbm.at[pl.ds((gid0 + g) * CH, CH)],
                                      ebuf[b], seme[b]).wait()

                @pl.when(gg >= 1)
                def _():
                    pltpu.make_async_copy(
                        msg[b], osh.at[idx_d.at[off + g]], semw[b]).wait()

                for i in range(CH):
                    albuf[pl.ds(i * W16, W16)] = ebuf[b][i, :] / dnr[b][i, :]

                @pl.loop(0, CH)
                def _edge(j):
                    av = albuf[pl.ds(j * W16, W16)]
                    for kq in range(4):
                        v = hrw[b][j, pl.ds(kq * 16, 16)]
                        me = jnp.full((16,), av[2 * kq], _f32)
                        mo = jnp.full((16,), av[2 * kq + 1], _f32)
                        msg[b][j, pl.ds(kq * 16, 16)] = (
                            v * jnp.where(iomask, me, mo))

                pltpu.async_copy(msg[b], osh.at[idx_d.at[off + g]], semw[b],
                                 add=True)

                @pl.when(g + 2 < nch)
                def _():
                    issue(g + 2, b)

        for b in range(2):
            g_last = nch - 2 + b
            pltpu.make_async_copy(
                msg[b], osh.at[idx_d.at[off + g_last]], semw[b]).wait()
        plsc.subcore_barrier()

        @pl.when(c == 0)
        def _():
            pltpu.sync_copy(osh.at[pl.ds(s * RS, RS)],
                            oa_hbm.at[pl.ds(s * RS, RS)])

        @pl.when(c == 1)
        def _():
            pltpu.sync_copy(osh.at[pl.ds(s * RS, RS)],
                            ob_hbm.at[pl.ds(s * RS, RS)])

    return k(src, dst, ee, da, db, h, z64)


def _edge_pass2_l2(src, dst, ee, da, db, h2, z48):
    mesh = plsc.VectorSubcoreMesh(**_MESH)

    @functools.partial(
        pl.kernel,
        out_type=[
            jax.ShapeDtypeStruct((NP, C_PAD), _f32),
            jax.ShapeDtypeStruct((NP, C_PAD), _f32),
        ],
        mesh=mesh,
        compiler_params=pltpu.CompilerParams(use_tc_tiling_on_sc=False),
        scratch_types=[
            pltpu.VMEM((NMXC2, CH2), _i32),
            pltpu.VMEM((NMXC2, CH2), _i32),
            pltpu.VMEM((CH2, W16), _f32),
            pltpu.VMEM((CH2, W16), _f32),
            pltpu.VMEM((CH2, W16), _f32),
            pltpu.VMEM((CH2, W16), _f32),
            pltpu.VMEM((CH2 * W16,), _f32),
            pltpu.VMEM((CH2, F2), _f32),
            pltpu.VMEM((CH2, F2), _f32),
            pltpu.VMEM((CH2, C_PAD), _f32),
            pltpu.VMEM((CH2, C_PAD), _f32),
            pltpu.VMEM((96,), _f32),
            pltpu.VMEM_SHARED((NP, C_PAD), _f32),
            pltpu.VMEM_SHARED((NP, W16), _f32),
            pltpu.SemaphoreType.DMA,
            pltpu.SemaphoreType.DMA,
            pltpu.SemaphoreType.DMA,
            pltpu.SemaphoreType.DMA,
            pltpu.SemaphoreType.DMA,
            pltpu.SemaphoreType.DMA,
            pltpu.SemaphoreType.DMA,
            pltpu.SemaphoreType.DMA,
        ],
    )
    def k(src_hbm, dst_hbm, ee_hbm, da_hbm, db_hbm, h_hbm, z_hbm,
          oa_hbm, ob_hbm, idx_s, idx_d, eb0, eb1, dn0, dn1, albuf,
          hr0, hr1, ms0, ms1, accbuf, osh, dsum,
          sh0, sh1, sd0, sd1, se0, se1, sw0, sw1):
        c = lax.axis_index("c")
        s = lax.axis_index("s")
        gid0 = jnp.where(c == 0, s * N0C2, NS * N0C2 + s * N1C2)
        nch = jnp.where(c == 0, N0C2, N1C2)
        start = jnp.minimum(gid0, TCHN2 - NMXC2)
        off = gid0 - start
        pltpu.sync_copy(src_hbm.at[pl.ds(start, NMXC2)], idx_s)
        pltpu.sync_copy(dst_hbm.at[pl.ds(start, NMXC2)], idx_d)
        pltpu.sync_copy(z_hbm.at[pl.ds(s * RS, RS)], osh.at[pl.ds(s * RS, RS)])
        for r in range(RS // CH2):
            row0 = s * RS + r * CH2
            pltpu.sync_copy(da_hbm.at[pl.ds(row0, CH2)], eb0)
            pltpu.sync_copy(db_hbm.at[pl.ds(row0, CH2)], eb1)
            for i in range(CH2):
                eb0[i, :] = eb0[i, :] + eb1[i, :] + 1e-16
            pltpu.sync_copy(eb0, dsum.at[pl.ds(row0, CH2)])
        accbuf[pl.ds(80, 16)] = jnp.zeros((16,), _f32)
        plsc.subcore_barrier()
        ebuf = [eb0, eb1]
        dnr = [dn0, dn1]
        hrw = [hr0, hr1]
        msg = [ms0, ms1]
        semh = [sh0, sh1]
        semd = [sd0, sd1]
        seme = [se0, se1]
        semw = [sw0, sw1]
        iomask = lax.iota(_i32, 16) < 8

        def issue(g, b):
            pltpu.async_copy(h_hbm.at[idx_s.at[off + g]], hrw[b], semh[b])
            pltpu.async_copy(dsum.at[idx_d.at[off + g]], dnr[b], semd[b])
            pltpu.async_copy(ee_hbm.at[pl.ds((gid0 + g) * CH2, CH2)],
                             ebuf[b], seme[b])

        issue(0, 0)
        issue(1, 1)

        @pl.loop(0, nch // 2)
        def _gg(gg):
            for b in range(2):
                g = gg * 2 + b
                pltpu.make_async_copy(h_hbm.at[idx_s.at[off + g]], hrw[b],
                                      semh[b]).wait()
                pltpu.make_async_copy(dsum.at[idx_d.at[off + g]], dnr[b],
                                      semd[b]).wait()
                pltpu.make_async_copy(ee_hbm.at[pl.ds((gid0 + g) * CH2, CH2)],
                                      ebuf[b], seme[b]).wait()

                @pl.when(gg >= 1)
                def _():
                    pltpu.make_async_copy(
                        msg[b], osh.at[idx_d.at[off + g]], semw[b]).wait()

                for i in range(CH2):
                    albuf[pl.ds(i * W16, W16)] = ebuf[b][i, :] / dnr[b][i, :]

                @pl.loop(0, CH2)
                def _edge(j):
                    av = albuf[pl.ds(j * W16, W16)]
                    acc = [jnp.zeros((16,), _f32) for _ in range(5)]
                    for p in range(4):
                        off = p * 80
                        me = jnp.full((16,), av[2 * p], _f32)
                        mo = jnp.full((16,), av[2 * p + 1], _f32)
                        mm = jnp.where(iomask, me, mo)
                        acc[0] = acc[0] + me * hrw[b][j, pl.ds(off, 16)]
                        acc[1] = acc[1] + me * hrw[b][j, pl.ds(off + 16, 16)]
                        acc[2] = acc[2] + mm * hrw[b][j, pl.ds(off + 32, 16)]
                        acc[3] = acc[3] + mo * hrw[b][j, pl.ds(off + 48, 16)]
                        acc[4] = acc[4] + mo * hrw[b][j, pl.ds(off + 64, 16)]
                    for q in range(5):
                        accbuf[pl.ds(q * 16, 16)] = acc[q]
                    msg[b][j, pl.ds(0, 16)] = (accbuf[pl.ds(0, 16)]
                                               + accbuf[pl.ds(40, 16)])
                    msg[b][j, pl.ds(16, 16)] = (accbuf[pl.ds(16, 16)]
                                                + accbuf[pl.ds(56, 16)])
                    msg[b][j, pl.ds(32, 16)] = (accbuf[pl.ds(32, 16)]
                                                + accbuf[pl.ds(72, 16)])

                pltpu.async_copy(msg[b], osh.at[idx_d.at[off + g]], semw[b],
                                 add=True)

                @pl.when(g + 2 < nch)
                def _():
                    issue(g + 2, b)

        for b in range(2):
            g_last = nch - 2 + b
            pltpu.make_async_copy(
                msg[b], osh.at[idx_d.at[off + g_last]], semw[b]).wait()
        plsc.subcore_barrier()

        @pl.when(c == 0)
        def _():
            pltpu.sync_copy(osh.at[pl.ds(s * RS, RS)],
                            oa_hbm.at[pl.ds(s * RS, RS)])

        @pl.when(c == 1)
        def _():
            pltpu.sync_copy(osh.at[pl.ds(s * RS, RS)],
                            ob_hbm.at[pl.ds(s * RS, RS)])

    return k(src, dst, ee, da, db, h2, z48)


_DOT = dict(preferred_element_type=_f32, precision=lax.Precision.HIGHEST)


def _tc1_body(x_ref, w_ref, as_ref, ad_ref, h_ref, s_ref, d_ref):
    h = jnp.dot(x_ref[...], w_ref[...], **_DOT)
    h_ref[...] = h
    a_s = jnp.dot(h, as_ref[...], **_DOT)
    a_d = jnp.dot(h, ad_ref[...], **_DOT)
    s_ref[...] = jnp.concatenate([a_s, a_d], axis=1)
    d_ref[...] = jnp.concatenate([a_d, a_s], axis=1)


def _tc1(x, w1, a_s, a_d):
    bn = 1024
    return pl.pallas_call(
        _tc1_body,
        grid=(NP // bn,),
        in_specs=[
            pl.BlockSpec((bn, D_IN), lambda i: (i, 0)),
            pl.BlockSpec((D_IN, F1), lambda i: (0, 0)),
            pl.BlockSpec((F1, HEADS), lambda i: (0, 0)),
            pl.BlockSpec((F1, HEADS), lambda i: (0, 0)),
        ],
        out_specs=[
            pl.BlockSpec((bn, F1), lambda i: (i, 0)),
            pl.BlockSpec((bn, W16), lambda i: (i, 0)),
            pl.BlockSpec((bn, W16), lambda i: (i, 0)),
        ],
        out_shape=[
            jax.ShapeDtypeStruct((NP, F1), _f32),
            jax.ShapeDtypeStruct((NP, W16), _f32),
            jax.ShapeDtypeStruct((NP, W16), _f32),
        ],
    )(x, w1, a_s, a_d)


def _tc2_body(oa_ref, ob_ref, b_ref, w_ref, as_ref, ad_ref,
              h_ref, s_ref, d_ref):
    t = oa_ref[...] + ob_ref[...] + b_ref[...]
    t = jnp.where(t > 0, t, jnp.exp(jnp.minimum(t, 0.0)) - 1.0)
    h = jnp.dot(t, w_ref[...], **_DOT)
    h_ref[...] = h
    a_s = jnp.dot(h, as_ref[...], **_DOT)
    a_d = jnp.dot(h, ad_ref[...], **_DOT)
    s_ref[...] = jnp.concatenate([a_s, a_d], axis=1)
    d_ref[...] = jnp.concatenate([a_d, a_s], axis=1)


def _tc2(oa, ob, b1, w2, a_s, a_d):
    bn = 1024
    return pl.pallas_call(
        _tc2_body,
        grid=(NP // bn,),
        in_specs=[
            pl.BlockSpec((bn, F1), lambda i: (i, 0)),
            pl.BlockSpec((bn, F1), lambda i: (i, 0)),
            pl.BlockSpec((1, F1), lambda i: (0, 0)),
            pl.BlockSpec((F1, F2), lambda i: (0, 0)),
            pl.BlockSpec((F2, HEADS), lambda i: (0, 0)),
            pl.BlockSpec((F2, HEADS), lambda i: (0, 0)),
        ],
        out_specs=[
            pl.BlockSpec((bn, F2), lambda i: (i, 0)),
            pl.BlockSpec((bn, W16), lambda i: (i, 0)),
            pl.BlockSpec((bn, W16), lambda i: (i, 0)),
        ],
        out_shape=[
            jax.ShapeDtypeStruct((NP, F2), _f32),
            jax.ShapeDtypeStruct((NP, W16), _f32),
            jax.ShapeDtypeStruct((NP, W16), _f32),
        ],
    )(oa, ob, b1, w2, a_s, a_d)


def _tc3_body(oa_ref, ob_ref, b_ref, o_ref):
    t = oa_ref[...] + ob_ref[...]
    o_ref[...] = t[:, :NUM_CLASSES] * (1.0 / OUT_HEADS) + b_ref[...]


def _tc3(oa, ob, b2):
    bn = 1000
    return pl.pallas_call(
        _tc3_body,
        grid=(N // bn,),
        in_specs=[
            pl.BlockSpec((bn, C_PAD), lambda i: (i, 0)),
            pl.BlockSpec((bn, C_PAD), lambda i: (i, 0)),
            pl.BlockSpec((1, NUM_CLASSES), lambda i: (0, 0)),
        ],
        out_specs=pl.BlockSpec((bn, NUM_CLASSES), lambda i: (i, 0)),
        out_shape=jax.ShapeDtypeStruct((N, NUM_CLASSES), _f32),
    )(oa, ob, b2)


def _expand_att(a, heads, ch):
    return jnp.repeat(jnp.eye(heads, dtype=_f32), ch, axis=0) * a.reshape(-1, 1)


def kernel(x, edge_index, W1, a_src1, a_dst1, b1, W2, a_src2, a_dst2, b2):
    src = edge_index[0].astype(_i32)
    dst = edge_index[1].astype(_i32)
    pad = jnp.full((EP - E,), N, _i32)
    src = jnp.concatenate([src, pad])
    dst = jnp.concatenate([dst, pad])
    src128 = src.reshape(EP // CH, CH)
    dst128 = dst.reshape(EP // CH, CH)
    src64 = src.reshape(EP // CH2, CH2)
    dst64 = dst.reshape(EP // CH2, CH2)

    xp = jnp.pad(x, ((0, NP - N), (0, 0)))
    as1 = _expand_att(a_src1, HEADS, HID)
    ad1 = _expand_att(a_dst1, HEADS, HID)
    as2 = _expand_att(a_src2, OUT_HEADS, NUM_CLASSES)
    ad2 = _expand_att(a_dst2, OUT_HEADS, NUM_CLASSES)

    z16 = jnp.zeros((NP, W16), _f32)
    z48 = jnp.zeros((NP, C_PAD), _f32)
    z64 = jnp.zeros((NP, F1), _f32)

    h1, ts1, td1 = _tc1(xp, W1, as1, ad1)
    ee1, da1, db1 = _edge_pass1(src128, dst128, ts1, td1, z16)
    oa1, ob1 = _edge_pass2_l1(src128, dst128, ee1, da1, db1, h1, z64)
    h2, ts2, td2 = _tc2(oa1, ob1, b1.reshape(1, F1), W2, as2, ad2)
    ee2, da2, db2 = _edge_pass1(src128, dst128, ts2, td2, z16)
    oa2, ob2 = _edge_pass2_l2(src64, dst64, ee2, da2, db2, h2, z48)
    return _tc3(oa2, ob2, b2.reshape(1, NUM_CLASSES))


# speedup vs baseline: 1.0829x; 1.0040x over previous
import functools

import jax
import jax.numpy as jnp
from jax import lax
from jax.experimental import pallas as pl
from jax.experimental.pallas import tpu as pltpu
from jax.experimental.pallas import tpu_sc as plsc

N = 10000
NP = 10240
E = 320000
EP = 327680
D_IN = 128
HEADS = 8
HID = 8
F1 = HEADS * HID
NUM_CLASSES = 40
OUT_HEADS = 8
F2 = OUT_HEADS * NUM_CLASSES
C_PAD = 48
W16 = 16

NC = 2
NS = 16
NT = NC * NS
EPT = EP // NT
CH = 128
NCHUNK = EPT // CH
CH2 = 64
NCHUNK2 = EPT // CH2
RS = NP // NS

N0C, N1C = 116, 44
NMXC = max(N0C, N1C)
TCHN = EP // CH
N0L1, N1L1 = 116, 44
N0C2, N1C2 = 232, 88
NMXC2 = max(N0C2, N1C2)
TCHN2 = EP // CH2

_MESH = dict(core_axis_name="c", subcore_axis_name="s", num_cores=NC,
             num_subcores=NS)

_f32 = jnp.float32
_i32 = jnp.int32


def _edge_pass1(src, dst, ts, td, z16):
    mesh = plsc.VectorSubcoreMesh(**_MESH)

    @functools.partial(
        pl.kernel,
        out_type=[
            jax.ShapeDtypeStruct((EP, W16), _f32),
            jax.ShapeDtypeStruct((NP, W16), _f32),
            jax.ShapeDtypeStruct((NP, W16), _f32),
        ],
        mesh=mesh,
        compiler_params=pltpu.CompilerParams(use_tc_tiling_on_sc=False),
        scratch_types=[
            pltpu.VMEM((NMXC, CH), _i32),
            pltpu.VMEM((NMXC, CH), _i32),
            pltpu.VMEM((CH, W16), _f32),
            pltpu.VMEM((CH, W16), _f32),
            pltpu.VMEM((CH, W16), _f32),
            pltpu.VMEM((CH, W16), _f32),
            pltpu.VMEM((CH, W16), _f32),
            pltpu.VMEM((CH, W16), _f32),
            pltpu.VMEM_SHARED((NP, W16), _f32),
            pltpu.SemaphoreType.DMA,
            pltpu.SemaphoreType.DMA,
            pltpu.SemaphoreType.DMA,
            pltpu.SemaphoreType.DMA,
            pltpu.SemaphoreType.DMA,
            pltpu.SemaphoreType.DMA,
            pltpu.SemaphoreType.DMA,
            pltpu.SemaphoreType.DMA,
        ],
    )
    def k(src_hbm, dst_hbm, ts_hbm, td_hbm, z_hbm, ee_hbm, da_hbm, db_hbm,
          idx_s, idx_d, sr0, sr1, dr0, dr1, eb0, eb1, dsh,
          ss0, ss1, sd0, sd1, se0, se1, sw0, sw1):
        c = lax.axis_index("c")
        s = lax.axis_index("s")
        gid0 = jnp.where(c == 0, s * N0C, NS * N0C + s * N1C)
        nch = jnp.where(c == 0, N0C, N1C)
        start = jnp.minimum(gid0, TCHN - NMXC)
        off = gid0 - start
        pltpu.sync_copy(src_hbm.at[pl.ds(start, NMXC)], idx_s)
        pltpu.sync_copy(dst_hbm.at[pl.ds(start, NMXC)], idx_d)
        pltpu.sync_copy(z_hbm.at[pl.ds(s * RS, RS)], dsh.at[pl.ds(s * RS, RS)])
        plsc.subcore_barrier()
        srows = [sr0, sr1]
        drows = [dr0, dr1]
        ebuf = [eb0, eb1]
        sems = [ss0, ss1]
        semd = [sd0, sd1]
        seme = [se0, se1]
        semw = [sw0, sw1]

        def issue(g, b):
            pltpu.async_copy(ts_hbm.at[idx_s.at[off + g]], srows[b], sems[b])
            pltpu.async_copy(td_hbm.at[idx_d.at[off + g]], drows[b], semd[b])

        issue(0, 0)
        issue(1, 1)

        @pl.loop(0, nch // 2)
        def _gg(gg):
            for b in range(2):
                g = gg * 2 + b
                pltpu.make_async_copy(ts_hbm.at[idx_s.at[off + g]], srows[b],
                                      sems[b]).wait()
                pltpu.make_async_copy(td_hbm.at[idx_d.at[off + g]], drows[b],
                                      semd[b]).wait()

                @pl.when(gg >= 1)
                def _():
                    pltpu.make_async_copy(
                        ebuf[b], ee_hbm.at[pl.ds((gid0 + g) * CH, CH)],
                        seme[b]).wait()
                    pltpu.make_async_copy(
                        ebuf[b], dsh.at[idx_d.at[off + g]], semw[b]).wait()

                for i in range(CH):
                    v = srows[b][i, :] + drows[b][i, :]
                    v = jnp.maximum(v, 0.2 * v)
                    ebuf[b][i, :] = jnp.exp(v)
                pltpu.async_copy(
                    ebuf[b], ee_hbm.at[pl.ds((gid0 + g) * CH, CH)], seme[b])
                pltpu.async_copy(
                    ebuf[b], dsh.at[idx_d.at[off + g]], semw[b], add=True)

                @pl.when(g + 2 < nch)
                def _():
                    issue(g + 2, b)

        for b in range(2):
            g_last = nch - 2 + b
            pltpu.make_async_copy(
                ebuf[b], ee_hbm.at[pl.ds((gid0 + g_last) * CH, CH)],
                seme[b]).wait()
            pltpu.make_async_copy(
                ebuf[b], dsh.at[idx_d.at[off + g_last]], semw[b]).wait()
        plsc.subcore_barrier()

        @pl.when(c == 0)
        def _():
            pltpu.sync_copy(dsh.at[pl.ds(s * RS, RS)],
                            da_hbm.at[pl.ds(s * RS, RS)])

        @pl.when(c == 1)
        def _():
            pltpu.sync_copy(dsh.at[pl.ds(s * RS, RS)],
                            db_hbm.at[pl.ds(s * RS, RS)])

    return k(src, dst, ts, td, z16)


def _edge_pass2_l1(src, dst, ee, da, db, h, z64):
    mesh = plsc.VectorSubcoreMesh(**_MESH)

    @functools.partial(
        pl.kernel,
        out_type=[
            jax.ShapeDtypeStruct((NP, F1), _f32),
            jax.ShapeDtypeStruct((NP, F1), _f32),
        ],
        mesh=mesh,
        compiler_params=pltpu.CompilerParams(use_tc_tiling_on_sc=False),
        scratch_types=[
            pltpu.VMEM((NMXC, CH), _i32),
            pltpu.VMEM((NMXC, CH), _i32),
            pltpu.VMEM((CH, W16), _f32),
            pltpu.VMEM((CH, W16), _f32),
            pltpu.VMEM((CH, W16), _f32),
            pltpu.VMEM((CH, W16), _f32),
            pltpu.VMEM((CH * W16,), _f32),
            pltpu.VMEM((CH, F1), _f32),
            pltpu.VMEM((CH, F1), _f32),
            pltpu.VMEM((CH, F1), _f32),
            pltpu.VMEM((CH, F1), _f32),
            pltpu.VMEM_SHARED((NP, F1), _f32),
            pltpu.VMEM_SHARED((NP, W16), _f32),
            pltpu.SemaphoreType.DMA,
            pltpu.SemaphoreType.DMA,
            pltpu.SemaphoreType.DMA,
            pltpu.SemaphoreType.DMA,
            pltpu.SemaphoreType.DMA,
            pltpu.SemaphoreType.DMA,
            pltpu.SemaphoreType.DMA,
            pltpu.SemaphoreType.DMA,
        ],
    )
    def k(src_hbm, dst_hbm, ee_hbm, da_hbm, db_hbm, h_hbm, z_hbm,
          oa_hbm, ob_hbm, idx_s, idx_d, eb0, eb1, dn0, dn1, albuf,
          hr0, hr1, ms0, ms1, osh, dsum,
          sh0, sh1, sd0, sd1, se0, se1, sw0, sw1):
        c = lax.axis_index("c")
        s = lax.axis_index("s")
        gid0 = jnp.where(c == 0, s * N0L1, NS * N0L1 + s * N1L1)
        nch = jnp.where(c == 0, N0L1, N1L1)
        start = jnp.minimum(gid0, TCHN - NMXC)
        off = gid0 - start
        pltpu.sync_copy(src_hbm.at[pl.ds(start, NMXC)], idx_s)
        pltpu.sync_copy(dst_hbm.at[pl.ds(start, NMXC)], idx_d)
        pltpu.sync_copy(z_hbm.at[pl.ds(s * RS, RS)], osh.at[pl.ds(s * RS, RS)])
        for r in range(RS // CH):
            row0 = s * RS + r * CH
            pltpu.sync_copy(da_hbm.at[pl.ds(row0, CH)], eb0)
            pltpu.sync_copy(db_hbm.at[pl.ds(row0, CH)], eb1)
            for i in range(CH):
                eb0[i, :] = eb0[i, :] + eb1[i, :] + 1e-16
            pltpu.sync_copy(eb0, dsum.at[pl.ds(row0, CH)])
        plsc.subcore_barrier()
        ebuf = [eb0, eb1]
        dnr = [dn0, dn1]
        hrw = [hr0, hr1]
        msg = [ms0, ms1]
        semh = [sh0, sh1]
        semd = [sd0, sd1]
        seme = [se0, se1]
        semw = [sw0, sw1]
        iomask = lax.iota(_i32, 16) < 8

        def issue(g, b):
            pltpu.async_copy(h_hbm.at[idx_s.at[off + g]], hrw[b], semh[b])
            pltpu.async_copy(dsum.at[idx_d.at[off + g]], dnr[b], semd[b])
            pltpu.async_copy(ee_hbm.at[pl.ds((gid0 + g) * CH, CH)],
                             ebuf[b], seme[b])

        issue(0, 0)
        issue(1, 1)

        @pl.loop(0, nch // 2)
        def _gg(gg):
            for b in range(2):
                g = gg * 2 + b
                pltpu.make_async_copy(h_hbm.at[idx_s.at[off + g]], hrw[b],
                                      semh[b]).wait()
                pltpu.make_async_copy(dsum.at[idx_d.at[off + g]], dnr[b],
                                      semd[b]).wait()
                pltpu.make_async_copy(ee_hbm.at[pl.ds((gid0 + g) * CH, CH)],
                                      ebuf[b], seme[b]).wait()

                @pl.when(gg >= 1)
                def _():
                    pltpu.make_async_copy(
                        msg[b], osh.at[idx_d.at[off + g]], semw[b]).wait()

                for i in range(CH):
                    albuf[pl.ds(i * W16, W16)] = ebuf[b][i, :] / dnr[b][i, :]

                @pl.loop(0, CH)
                def _edge(j):
                    av = albuf[pl.ds(j * W16, W16)]
                    for kq in range(4):
                        v = hrw[b][j, pl.ds(kq * 16, 16)]
                        me = jnp.full((16,), av[2 * kq], _f32)
                        mo = jnp.full((16,), av[2 * kq + 1], _f32)
                        msg[b][j, pl.ds(kq * 16, 16)] = (
                            v * jnp.where(iomask, me, mo))

                pltpu.async_copy(msg[b], osh.at[idx_d.at[off + g]], semw[b],
                                 add=True)

                @pl.when(g + 2 < nch)
                def _():
                    issue(g + 2, b)

        for b in range(2):
            g_last = nch - 2 + b
            pltpu.make_async_copy(
                msg[b], osh.at[idx_d.at[off + g_last]], semw[b]).wait()
        plsc.subcore_barrier()

        @pl.when(c == 0)
        def _():
            pltpu.sync_copy(osh.at[pl.ds(s * RS, RS)],
                            oa_hbm.at[pl.ds(s * RS, RS)])

        @pl.when(c == 1)
        def _():
            pltpu.sync_copy(osh.at[pl.ds(s * RS, RS)],
                            ob_hbm.at[pl.ds(s * RS, RS)])

    return k(src, dst, ee, da, db, h, z64)


def _edge_pass2_l2(src, dst, ee, da, db, h2, z48):
    mesh = plsc.VectorSubcoreMesh(**_MESH)

    @functools.partial(
        pl.kernel,
        out_type=[
            jax.ShapeDtypeStruct((NP, C_PAD), _f32),
            jax.ShapeDtypeStruct((NP, C_PAD), _f32),
        ],
        mesh=mesh,
        compiler_params=pltpu.CompilerParams(use_tc_tiling_on_sc=False),
        scratch_types=[
            pltpu.VMEM((NMXC2, CH2), _i32),
            pltpu.VMEM((NMXC2, CH2), _i32),
            pltpu.VMEM((CH2, W16), _f32),
            pltpu.VMEM((CH2, W16), _f32),
            pltpu.VMEM((CH2, W16), _f32),
            pltpu.VMEM((CH2, W16), _f32),
            pltpu.VMEM((CH2 * W16,), _f32),
            pltpu.VMEM((CH2, F2), _f32),
            pltpu.VMEM((CH2, F2), _f32),
            pltpu.VMEM((CH2, C_PAD), _f32),
            pltpu.VMEM((CH2, C_PAD), _f32),
            pltpu.VMEM((96,), _f32),
            pltpu.VMEM_SHARED((NP, C_PAD), _f32),
            pltpu.VMEM_SHARED((NP, W16), _f32),
            pltpu.SemaphoreType.DMA,
            pltpu.SemaphoreType.DMA,
            pltpu.SemaphoreType.DMA,
            pltpu.SemaphoreType.DMA,
            pltpu.SemaphoreType.DMA,
            pltpu.SemaphoreType.DMA,
            pltpu.SemaphoreType.DMA,
            pltpu.SemaphoreType.DMA,
        ],
    )
    def k(src_hbm, dst_hbm, ee_hbm, da_hbm, db_hbm, h_hbm, z_hbm,
          oa_hbm, ob_hbm, idx_s, idx_d, eb0, eb1, dn0, dn1, albuf,
          hr0, hr1, ms0, ms1, accbuf, osh, dsum,
          sh0, sh1, sd0, sd1, se0, se1, sw0, sw1):
        c = lax.axis_index("c")
        s = lax.axis_index("s")
        gid0 = jnp.where(c == 0, s * N0C2, NS * N0C2 + s * N1C2)
        nch = jnp.where(c == 0, N0C2, N1C2)
        start = jnp.minimum(gid0, TCHN2 - NMXC2)
        off = gid0 - start
        pltpu.sync_copy(src_hbm.at[pl.ds(start, NMXC2)], idx_s)
        pltpu.sync_copy(dst_hbm.at[pl.ds(start, NMXC2)], idx_d)
        pltpu.sync_copy(z_hbm.at[pl.ds(s * RS, RS)], osh.at[pl.ds(s * RS, RS)])
        for r in range(RS // CH2):
            row0 = s * RS + r * CH2
            pltpu.sync_copy(da_hbm.at[pl.ds(row0, CH2)], eb0)
            pltpu.sync_copy(db_hbm.at[pl.ds(row0, CH2)], eb1)
            for i in range(CH2):
                eb0[i, :] = eb0[i, :] + eb1[i, :] + 1e-16
            pltpu.sync_copy(eb0, dsum.at[pl.ds(row0, CH2)])
        accbuf[pl.ds(80, 16)] = jnp.zeros((16,), _f32)
        plsc.subcore_barrier()
        ebuf = [eb0, eb1]
        dnr = [dn0, dn1]
        hrw = [hr0, hr1]
        msg = [ms0, ms1]
        semh = [sh0, sh1]
        semd = [sd0, sd1]
        seme = [se0, se1]
        semw = [sw0, sw1]
        iomask = lax.iota(_i32, 16) < 8

        def issue(g, b):
            pltpu.async_copy(h_hbm.at[idx_s.at[off + g]], hrw[b], semh[b])
            pltpu.async_copy(dsum.at[idx_d.at[off + g]], dnr[b], semd[b])
            pltpu.async_copy(ee_hbm.at[pl.ds((gid0 + g) * CH2, CH2)],
                             ebuf[b], seme[b])

        issue(0, 0)
        issue(1, 1)

        @pl.loop(0, nch // 2)
        def _gg(gg):
            for b in range(2):
                g = gg * 2 + b
                pltpu.make_async_copy(h_hbm.at[idx_s.at[off + g]], hrw[b],
                                      semh[b]).wait()
                pltpu.make_async_copy(dsum.at[idx_d.at[off + g]], dnr[b],
                                      semd[b]).wait()
                pltpu.make_async_copy(ee_hbm.at[pl.ds((gid0 + g) * CH2, CH2)],
                                      ebuf[b], seme[b]).wait()

                @pl.when(gg >= 1)
                def _():
                    pltpu.make_async_copy(
                        msg[b], osh.at[idx_d.at[off + g]], semw[b]).wait()

                for i in range(CH2):
                    albuf[pl.ds(i * W16, W16)] = ebuf[b][i, :] / dnr[b][i, :]

                @pl.loop(0, CH2)
                def _edge(j):
                    av = albuf[pl.ds(j * W16, W16)]
                    acc = [jnp.zeros((16,), _f32) for _ in range(5)]
                    for p in range(4):
                        off = p * 80
                        me = jnp.full((16,), av[2 * p], _f32)
                        mo = jnp.full((16,), av[2 * p + 1], _f32)
                        mm = jnp.where(iomask, me, mo)
                        acc[0] = acc[0] + me * hrw[b][j, pl.ds(off, 16)]
                        acc[1] = acc[1] + me * hrw[b][j, pl.ds(off + 16, 16)]
                        acc[2] = acc[2] + mm * hrw[b][j, pl.ds(off + 32, 16)]
                        acc[3] = acc[3] + mo * hrw[b][j, pl.ds(off + 48, 16)]
                        acc[4] = acc[4] + mo * hrw[b][j, pl.ds(off + 64, 16)]
                    for q in range(5):
                        accbuf[pl.ds(q * 16, 16)] = acc[q]
                    msg[b][j, pl.ds(0, 16)] = (accbuf[pl.ds(0, 16)]
                                               + accbuf[pl.ds(40, 16)])
                    msg[b][j, pl.ds(16, 16)] = (accbuf[pl.ds(16, 16)]
                                                + accbuf[pl.ds(56, 16)])
                    msg[b][j, pl.ds(32, 16)] = (accbuf[pl.ds(32, 16)]
                                                + accbuf[pl.ds(72, 16)])

                pltpu.async_copy(msg[b], osh.at[idx_d.at[off + g]], semw[b],
                                 add=True)

                @pl.when(g + 2 < nch)
                def _():
                    issue(g + 2, b)

        for b in range(2):
            g_last = nch - 2 + b
            pltpu.make_async_copy(
                msg[b], osh.at[idx_d.at[off + g_last]], semw[b]).wait()
        plsc.subcore_barrier()

        @pl.when(c == 0)
        def _():
            pltpu.sync_copy(osh.at[pl.ds(s * RS, RS)],
                            oa_hbm.at[pl.ds(s * RS, RS)])

        @pl.when(c == 1)
        def _():
            pltpu.sync_copy(osh.at[pl.ds(s * RS, RS)],
                            ob_hbm.at[pl.ds(s * RS, RS)])

    return k(src, dst, ee, da, db, h2, z48)


_DOT = dict(preferred_element_type=_f32, precision=lax.Precision.HIGHEST)


def _tc1_body(x_ref, w_ref, as_ref, ad_ref, h_ref, s_ref, d_ref):
    h = jnp.dot(x_ref[...], w_ref[...], **_DOT)
    h_ref[...] = h
    a_s = jnp.dot(h, as_ref[...], **_DOT)
    a_d = jnp.dot(h, ad_ref[...], **_DOT)
    s_ref[...] = jnp.concatenate([a_s, a_d], axis=1)
    d_ref[...] = jnp.concatenate([a_d, a_s], axis=1)


def _tc1(x, w1, a_s, a_d):
    bn = 1024
    return pl.pallas_call(
        _tc1_body,
        grid=(NP // bn,),
        in_specs=[
            pl.BlockSpec((bn, D_IN), lambda i: (i, 0)),
            pl.BlockSpec((D_IN, F1), lambda i: (0, 0)),
            pl.BlockSpec((F1, HEADS), lambda i: (0, 0)),
            pl.BlockSpec((F1, HEADS), lambda i: (0, 0)),
        ],
        out_specs=[
            pl.BlockSpec((bn, F1), lambda i: (i, 0)),
            pl.BlockSpec((bn, W16), lambda i: (i, 0)),
            pl.BlockSpec((bn, W16), lambda i: (i, 0)),
        ],
        out_shape=[
            jax.ShapeDtypeStruct((NP, F1), _f32),
            jax.ShapeDtypeStruct((NP, W16), _f32),
            jax.ShapeDtypeStruct((NP, W16), _f32),
        ],
    )(x, w1, a_s, a_d)


def _tc2_body(oa_ref, ob_ref, b_ref, w_ref, as_ref, ad_ref,
              h_ref, s_ref, d_ref):
    t = oa_ref[...] + ob_ref[...] + b_ref[...]
    t = jnp.where(t > 0, t, jnp.exp(jnp.minimum(t, 0.0)) - 1.0)
    h = jnp.dot(t, w_ref[...], **_DOT)
    h_ref[...] = h
    a_s = jnp.dot(h, as_ref[...], **_DOT)
    a_d = jnp.dot(h, ad_ref[...], **_DOT)
    s_ref[...] = jnp.concatenate([a_s, a_d], axis=1)
    d_ref[...] = jnp.concatenate([a_d, a_s], axis=1)


def _tc2(oa, ob, b1, w2, a_s, a_d):
    bn = 1024
    return pl.pallas_call(
        _tc2_body,
        grid=(NP // bn,),
        in_specs=[
            pl.BlockSpec((bn, F1), lambda i: (i, 0)),
            pl.BlockSpec((bn, F1), lambda i: (i, 0)),
            pl.BlockSpec((1, F1), lambda i: (0, 0)),
            pl.BlockSpec((F1, F2), lambda i: (0, 0)),
            pl.BlockSpec((F2, HEADS), lambda i: (0, 0)),
            pl.BlockSpec((F2, HEADS), lambda i: (0, 0)),
        ],
        out_specs=[
            pl.BlockSpec((bn, F2), lambda i: (i, 0)),
            pl.BlockSpec((bn, W16), lambda i: (i, 0)),
            pl.BlockSpec((bn, W16), lambda i: (i, 0)),
        ],
        out_shape=[
            jax.ShapeDtypeStruct((NP, F2), _f32),
            jax.ShapeDtypeStruct((NP, W16), _f32),
            jax.ShapeDtypeStruct((NP, W16), _f32),
        ],
    )(oa, ob, b1, w2, a_s, a_d)


def _tc3_body(oa_ref, ob_ref, b_ref, o_ref):
    t = oa_ref[...] + ob_ref[...]
    o_ref[...] = t[:, :NUM_CLASSES] * (1.0 / OUT_HEADS) + b_ref[...]


def _tc3(oa, ob, b2):
    bn = 1000
    return pl.pallas_call(
        _tc3_body,
        grid=(N // bn,),
        in_specs=[
            pl.BlockSpec((bn, C_PAD), lambda i: (i, 0)),
            pl.BlockSpec((bn, C_PAD), lambda i: (i, 0)),
            pl.BlockSpec((1, NUM_CLASSES), lambda i: (0, 0)),
        ],
        out_specs=pl.BlockSpec((bn, NUM_CLASSES), lambda i: (i, 0)),
        out_shape=jax.ShapeDtypeStruct((N, NUM_CLASSES), _f32),
    )(oa, ob, b2)


def _expand_att(a, heads, ch):
    return jnp.repeat(jnp.eye(heads, dtype=_f32), ch, axis=0) * a.reshape(-1, 1)


def kernel(x, edge_index, W1, a_src1, a_dst1, b1, W2, a_src2, a_dst2, b2):
    src = edge_index[0].astype(_i32)
    dst = edge_index[1].astype(_i32)
    pad = jnp.full((EP - E,), N, _i32)
    src = jnp.concatenate([src, pad])
    dst = jnp.concatenate([dst, pad])
    src128 = src.reshape(EP // CH, CH)
    dst128 = dst.reshape(EP // CH, CH)
    src64 = src.reshape(EP // CH2, CH2)
    dst64 = dst.reshape(EP // CH2, CH2)

    xp = jnp.pad(x, ((0, NP - N), (0, 0)))
    as1 = _expand_att(a_src1, HEADS, HID)
    ad1 = _expand_att(a_dst1, HEADS, HID)
    as2 = _expand_att(a_src2, OUT_HEADS, NUM_CLASSES)
    ad2 = _expand_att(a_dst2, OUT_HEADS, NUM_CLASSES)

    z16 = jnp.zeros((NP, W16), _f32)
    z48 = jnp.zeros((NP, C_PAD), _f32)
    z64 = jnp.zeros((NP, F1), _f32)

    h1, ts1, td1 = _tc1(xp, W1, as1, ad1)
    ee1, da1, db1 = _edge_pass1(src128, dst128, ts1, td1, z16)
    oa1, ob1 = _edge_pass2_l1(src128, dst128, ee1, da1, db1, h1, z64)
    h2, ts2, td2 = _tc2(oa1, ob1, b1.reshape(1, F1), W2, as2, ad2)
    ee2, da2, db2 = _edge_pass1(src128, dst128, ts2, td2, z16)
    oa2, ob2 = _edge_pass2_l2(src64, dst64, ee2, da2, db2, h2, z48)
    return _tc3(oa2, ob2, b2.reshape(1, NUM_CLASSES))
